# Initial kernel scaffold; baseline (speedup 1.0000x reference)
import jax
import jax.numpy as jnp
from jax.experimental import pallas as pl


def kernel(z, edge_index, W1, b1, W2, b2, Wg, bg, Wa, att_src, att_dst, ba):
    raise NotImplementedError("write your pallas kernel here")


# speedup vs baseline: 7.8518x; 7.8518x over previous
import functools
import jax
import jax.numpy as jnp
from jax import lax
from jax.experimental import pallas as pl
from jax.experimental.pallas import tpu as pltpu, tpu_sc as plsc

N = 10000
NPAD = 10240
E = 160000
H = 2
C = 512

NC = 2
NS = 16
NW = NC * NS
OWN = NPAD // NW
ECH = 2000
NCHUNK = E // ECH
STRIPE = E // NS
SCHUNK = STRIPE // ECH

GCN_CH = 2560
GCN_PASS = NPAD // NC // GCN_CH
GAT_CH = 1024
GAT_PASS = NPAD // NC // GAT_CH
GB_GCN = 32
GB_GAT = 16

_mesh = functools.partial(
    plsc.VectorSubcoreMesh, core_axis_name="c", subcore_axis_name="s"
)
_SC_PARAMS = pltpu.CompilerParams(needs_layout_passes=False)

_f32 = jnp.float32
_i32 = jnp.int32


def _iota16():
    return lax.iota(_i32, 16)


def _lrelu(v):
    return jnp.where(v >= 0.0, v, 0.2 * v)


def _shift_right(v, d, fill):
    i16 = _iota16()
    idx = jnp.maximum(i16 - d, 0)
    g = jnp.take_along_axis(v, idx, axis=0, mode="promise_in_bounds")
    return jnp.where(i16 >= d, g, fill)


def _shift_left1(v, fill):
    i16 = _iota16()
    idx = jnp.minimum(i16 + 1, 15)
    g = jnp.take_along_axis(v, idx, axis=0, mode="promise_in_bounds")
    return jnp.where(i16 <= 14, g, fill)


@functools.partial(
    pl.kernel,
    out_type=jax.ShapeDtypeStruct((NPAD,), _f32),
    mesh=_mesh(),
    compiler_params=_SC_PARAMS,
    scratch_types=[
        pltpu.VMEM((ECH,), _i32),
        pltpu.VMEM((OWN + 16,), _f32),
    ],
)
def _s1_deg(col_hbm, deg_hbm, colb, deg):
    w = lax.axis_index("s") * NC + lax.axis_index("c")
    base = w * OWN
    for i in range(OWN // 16):
        deg[pl.ds(i * 16, 16)] = jnp.full((16,), 1.0, _f32)
    deg[pl.ds(OWN, 16)] = jnp.zeros((16,), _f32)
    ones = jnp.full((16,), 1.0, _f32)

    def chunk_body(ch, _):
        pltpu.sync_copy(col_hbm.at[pl.ds(ch * ECH, ECH)], colb)

        def vec_body(v, _):
            c16 = colb[pl.ds(v * 16, 16)]
            m = (c16 >= base) & (c16 < base + OWN)
            lc = jnp.where(m, c16 - base, OWN)
            plsc.addupdate_scatter(deg, [lc], ones, mask=m)
            return 0

        lax.fori_loop(0, ECH // 16, vec_body, 0)
        return 0

    lax.fori_loop(0, NCHUNK, chunk_body, 0)
    pltpu.sync_copy(deg.at[pl.ds(0, OWN)], deg_hbm.at[pl.ds(base, OWN)])


SLAB_G = C // 128


@functools.partial(
    pl.kernel,
    out_type=jax.ShapeDtypeStruct((NPAD * SLAB_G, 128), _f32),
    mesh=_mesh(),
    compiler_params=_SC_PARAMS,
    scratch_types=[
        pltpu.VMEM_SHARED((GCN_CH * SLAB_G + 8, 128), _f32),
        pltpu.VMEM((ECH,), _i32),
        pltpu.VMEM((ECH,), _i32),
        pltpu.VMEM((ECH + 4 * GB_GCN,), _i32),
        pltpu.VMEM((ECH + 4 * GB_GCN,), _i32),
        pltpu.VMEM((GB_GCN * SLAB_G, 128), _f32),
        pltpu.VMEM((32, 128), _f32),
        pltpu.VMEM((GB_GCN * SLAB_G,), _i32),
        pltpu.VMEM((GB_GCN * SLAB_G,), _i32),
        pltpu.SemaphoreType.DMA,
    ],
)
def _s2_gcn(xgs_hbm, row_hbm, col_hbm, agg_hbm,
            acc, colb, rowb, frow, flc, gbuf, zbuf, brow, bidx, sem):
    core = lax.axis_index("c")
    s = lax.axis_index("s")
    for i in range(32):
        for j in range(8):
            zbuf[i, pl.ds(j * 16, 16)] = jnp.zeros((16,), _f32)

    for p in range(GCN_PASS):
        lo = core * (NPAD // NC) + p * GCN_CH
        rows_per = GCN_CH // NS * SLAB_G
        for i in range(rows_per // 32):
            pltpu.sync_copy(zbuf, acc.at[pl.ds(s * rows_per + i * 32, 32)])

        @pl.when(s == 0)
        def _():
            pltpu.sync_copy(zbuf.at[pl.ds(0, 8)],
                            acc.at[pl.ds(GCN_CH * SLAB_G, 8)])

        plsc.subcore_barrier()

        def chunk_body(ci, _):
            ebase = s * STRIPE + ci * ECH
            pltpu.sync_copy(col_hbm.at[pl.ds(ebase, ECH)], colb)
            pltpu.sync_copy(row_hbm.at[pl.ds(ebase, ECH)], rowb)

            def vec_body(v, cnt):
                c16 = colb[pl.ds(v * 16, 16)]
                r16 = rowb[pl.ds(v * 16, 16)]
                m = (c16 >= lo) & (c16 < lo + GCN_CH)
                plsc.store_compressed(frow.at[pl.ds(cnt, 16)], r16, mask=m)
                plsc.store_compressed(
                    flc.at[pl.ds(cnt, 16)], c16 - lo, mask=m)
                return cnt + plsc.all_reduce_population_count(m)[0]

            cnt = lax.fori_loop(0, ECH // 16, vec_body, _i32(0))
            for t in range(GB_GCN // 16):
                frow[pl.ds(cnt + t * 16, 16)] = jnp.zeros((16,), _i32)
                flc[pl.ds(cnt + t * 16, 16)] = jnp.full(
                    (16,), GCN_CH, _i32)
            nb = (cnt + GB_GCN - 1) // GB_GCN

            def flush_body(b, _):
                for g in range(GB_GCN // 16):
                    r16 = frow[pl.ds(b * GB_GCN + g * 16, 16)]
                    l16 = flc[pl.ds(b * GB_GCN + g * 16, 16)]
                    for j in range(SLAB_G):
                        brow[pl.ds(j * GB_GCN + g * 16, 16)] = (
                            r16 * SLAB_G + j)
                        bidx[pl.ds(j * GB_GCN + g * 16, 16)] = (
                            l16 * SLAB_G + j)
                pltpu.async_copy(xgs_hbm.at[brow], gbuf, sem).wait()
                pltpu.sync_copy(gbuf, acc.at[bidx], add=True)
                return 0

            lax.fori_loop(0, nb, flush_body, 0)
            return 0

        lax.fori_loop(0, SCHUNK, chunk_body, 0)
        plsc.subcore_barrier()
        pltpu.sync_copy(
            acc.at[pl.ds(s * rows_per, rows_per)],
            agg_hbm.at[pl.ds(lo * SLAB_G + s * rows_per, rows_per)])
        plsc.subcore_barrier()


_S3OWN = OWN * 2


@functools.partial(
    pl.kernel,
    out_type=(
        jax.ShapeDtypeStruct((NPAD * 2,), _f32),
        jax.ShapeDtypeStruct((NPAD * 2,), _f32),
        jax.ShapeDtypeStruct((NPAD * 2,), _f32),
    ),
    mesh=_mesh(),
    compiler_params=_SC_PARAMS,
    scratch_types=[
        pltpu.VMEM((NPAD * 2,), _f32),
        pltpu.VMEM((_S3OWN,), _f32),
        pltpu.VMEM((_S3OWN + 16,), _f32),
        pltpu.VMEM((_S3OWN,), _f32),
        pltpu.VMEM((_S3OWN + 16,), _f32),
        pltpu.VMEM((ECH,), _i32),
        pltpu.VMEM((ECH,), _i32),
        pltpu.VMEM((ECH + 32,), _i32),
        pltpu.VMEM((16,), _i32),
    ],
)
def _s3_stats(row_hbm, col_hbm, asrc_hbm, adst_hbm,
              amax_hbm, den_hbm, wself_hbm,
              asrc, adst, amax, aself, den, colb, rowb, fpk, cntb):
    w = lax.axis_index("s") * NC + lax.axis_index("c")
    base = w * OWN
    pltpu.sync_copy(asrc_hbm, asrc)
    pltpu.sync_copy(adst_hbm.at[pl.ds(base * 2, _S3OWN)], adst)
    for i in range(_S3OWN // 16):
        gidx = base * 2 + i * 16 + _iota16()
        a = plsc.load_gather(asrc, [gidx])
        b = adst[pl.ds(i * 16, 16)]
        v = _lrelu(a + b)
        aself[pl.ds(i * 16, 16)] = v
        amax[pl.ds(i * 16, 16)] = v
    amax[pl.ds(_S3OWN, 16)] = jnp.zeros((16,), _f32)

    def filter_chunk(ci, cnt_in):
        pltpu.sync_copy(col_hbm.at[pl.ds(ci * ECH, ECH)], colb)
        pltpu.sync_copy(row_hbm.at[pl.ds(ci * ECH, ECH)], rowb)

        def vec_body(v, cnt):
            c16 = colb[pl.ds(v * 16, 16)]
            r16 = rowb[pl.ds(v * 16, 16)]
            m = (c16 >= base) & (c16 < base + OWN)
            pk = r16 | ((c16 - base) << 14)
            plsc.store_compressed(fpk.at[pl.ds(cnt, 16)], pk, mask=m)
            return cnt + plsc.all_reduce_population_count(m)[0]

        return lax.fori_loop(0, ECH // 16, vec_body, cnt_in)

    def alpha16(g):
        pk = fpk[pl.ds(g * 16, 16)]
        r16 = pk & 16383
        lc16 = pk >> 14
        res = []
        for h in range(H):
            a = plsc.load_gather(asrc, [r16 * 2 + h])
            b = plsc.load_gather(adst, [jnp.minimum(lc16 * 2 + h,
                                                    _S3OWN - 1)])
            res.append((_lrelu(a + b), lc16 * 2 + h))
        return res

    def scanA(ci, _):
        cnt = filter_chunk(ci, _i32(0))
        fpk[pl.ds(cnt, 16)] = jnp.full((16,), (OWN << 14), _i32)
        ng = (cnt + 15) // 16

        def grp(g, _):
            for al, key in alpha16(g):
                key = jnp.minimum(key, _S3OWN)
                sk, sv = plsc.sort_key_val(key, al)
                fl = (sk != _shift_right(sk, 1, _i32(-1))).astype(_i32)
                mv = sv
                flc = fl
                for d in (1, 2, 4, 8):
                    mvs = _shift_right(mv, d, _f32(-1e30))
                    fls = _shift_right(flc, d, _i32(1))
                    mv = jnp.where(flc > 0, mv, jnp.maximum(mv, mvs))
                    flc = jnp.maximum(flc, fls)
                is_last = (sk != _shift_left1(sk, _i32(-1))) | (
                    _iota16() == 15)
                old = plsc.load_gather(amax, [sk])
                plsc.store_scatter(
                    amax, [sk], jnp.maximum(old, mv), mask=is_last)
            return 0

        lax.fori_loop(0, ng, grp, 0)
        return 0

    lax.fori_loop(0, NCHUNK, scanA, 0)

    for i in range(_S3OWN // 16):
        v = jnp.exp(aself[pl.ds(i * 16, 16)] - amax[pl.ds(i * 16, 16)])
        aself[pl.ds(i * 16, 16)] = v
        den[pl.ds(i * 16, 16)] = v
    den[pl.ds(_S3OWN, 16)] = jnp.zeros((16,), _f32)

    def scanB(ci, _):
        cnt = filter_chunk(ci, _i32(0))
        fpk[pl.ds(cnt, 16)] = jnp.full((16,), (OWN << 14), _i32)
        ng = (cnt + 15) // 16

        def grp(g, _):
            for al, key in alpha16(g):
                key = jnp.minimum(key, _S3OWN)
                mx = plsc.load_gather(amax, [key])
                wv = jnp.exp(al - mx)
                plsc.addupdate_scatter(den, [key], wv)
            return 0

        lax.fori_loop(0, ng, grp, 0)
        return 0

    lax.fori_loop(0, NCHUNK, scanB, 0)

    pltpu.sync_copy(amax.at[pl.ds(0, _S3OWN)],
                    amax_hbm.at[pl.ds(base * 2, _S3OWN)])
    pltpu.sync_copy(den.at[pl.ds(0, _S3OWN)],
                    den_hbm.at[pl.ds(base * 2, _S3OWN)])
    pltpu.sync_copy(aself, wself_hbm.at[pl.ds(base * 2, _S3OWN)])


SLAB_H = H * C // 128


@functools.partial(
    pl.kernel,
    out_type=jax.ShapeDtypeStruct((NPAD * SLAB_H, 128), _f32),
    mesh=_mesh(),
    compiler_params=_SC_PARAMS,
    scratch_types=[
        pltpu.VMEM_SHARED((GAT_CH * SLAB_H + 8, 128), _f32),
        pltpu.VMEM((NPAD * 2,), _f32),
        pltpu.VMEM((GAT_CH * 2,), _f32),
        pltpu.VMEM((GAT_CH * 2,), _f32),
        pltpu.VMEM((ECH,), _i32),
        pltpu.VMEM((ECH,), _i32),
        pltpu.VMEM((ECH + 4 * GB_GAT,), _i32),
        pltpu.VMEM((ECH + 4 * GB_GAT,), _i32),
        pltpu.VMEM((GB_GAT * SLAB_H, 128), _f32),
        pltpu.VMEM((GB_GAT,), _f32),
        pltpu.VMEM((GB_GAT,), _f32),
        pltpu.VMEM((32, 128), _f32),
        pltpu.VMEM((GB_GAT * SLAB_H,), _i32),
        pltpu.VMEM((GB_GAT * SLAB_H,), _i32),
        pltpu.SemaphoreType.DMA,
    ],
)
def _s4_gat(xh_hbm, row_hbm, col_hbm, asrc_hbm, adst_hbm, amax_hbm,
            num_hbm, acc, asrc, adst, amx, colb, rowb, frow, flc,
            gbuf, w0b, w1b, zbuf, brow, bidx, sem):
    core = lax.axis_index("c")
    s = lax.axis_index("s")
    pltpu.sync_copy(asrc_hbm, asrc)
    for i in range(32):
        for j in range(8):
            zbuf[i, pl.ds(j * 16, 16)] = jnp.zeros((16,), _f32)

    for p in range(GAT_PASS):
        lo = core * (NPAD // NC) + p * GAT_CH
        pltpu.sync_copy(adst_hbm.at[pl.ds(lo * 2, GAT_CH * 2)], adst)
        pltpu.sync_copy(amax_hbm.at[pl.ds(lo * 2, GAT_CH * 2)], amx)
        rows_per = GAT_CH // NS * SLAB_H
        for i in range(rows_per // 32):
            pltpu.sync_copy(zbuf, acc.at[pl.ds(s * rows_per + i * 32, 32)])

        @pl.when(s == 0)
        def _():
            pltpu.sync_copy(zbuf.at[pl.ds(0, 8)],
                            acc.at[pl.ds(GAT_CH * SLAB_H, 8)])

        plsc.subcore_barrier()

        def chunk_body(ci, _):
            ebase = s * STRIPE + ci * ECH
            pltpu.sync_copy(col_hbm.at[pl.ds(ebase, ECH)], colb)
            pltpu.sync_copy(row_hbm.at[pl.ds(ebase, ECH)], rowb)

            def vec_body(v, cnt):
                c16 = colb[pl.ds(v * 16, 16)]
                r16 = rowb[pl.ds(v * 16, 16)]
                m = (c16 >= lo) & (c16 < lo + GAT_CH)
                plsc.store_compressed(frow.at[pl.ds(cnt, 16)], r16, mask=m)
                plsc.store_compressed(
                    flc.at[pl.ds(cnt, 16)], c16 - lo, mask=m)
                return cnt + plsc.all_reduce_population_count(m)[0]

            cnt = lax.fori_loop(0, ECH // 16, vec_body, _i32(0))
            frow[pl.ds(cnt, 16)] = jnp.zeros((16,), _i32)
            flc[pl.ds(cnt, 16)] = jnp.full((16,), GAT_CH, _i32)
            nb = (cnt + GB_GAT - 1) // GB_GAT

            def flush_body(b, _):
                r16 = frow[pl.ds(b * GB_GAT, 16)]
                l16 = flc[pl.ds(b * GB_GAT, 16)]
                for j in range(SLAB_H):
                    brow[pl.ds(j * GB_GAT, 16)] = r16 * SLAB_H + j
                    bidx[pl.ds(j * GB_GAT, 16)] = l16 * SLAB_H + j
                pltpu.async_copy(xh_hbm.at[brow], gbuf, sem).wait()
                lidx = jnp.minimum(l16 * 2, GAT_CH * 2 - 2)
                for h, wb in ((0, w0b), (1, w1b)):
                    a = plsc.load_gather(asrc, [r16 * 2 + h])
                    bdd = plsc.load_gather(adst, [lidx + h])
                    mx = plsc.load_gather(amx, [lidx + h])
                    wb[...] = jnp.exp(_lrelu(a + bdd) - mx)

                def scale_row(i, _):
                    ri0 = jnp.full((16,), i, _i32)
                    s0 = plsc.load_gather(w0b, [ri0])
                    s1 = plsc.load_gather(w1b, [ri0])
                    i16 = _iota16()
                    for j in range(SLAB_H):
                        rj = jnp.full((16,), j * GB_GAT + i, _i32)
                        sc = s0 if j < SLAB_H // 2 else s1
                        for k in range(8):
                            ck = k * 16 + i16
                            v = plsc.load_gather(gbuf, [rj, ck])
                            plsc.store_scatter(gbuf, [rj, ck], v * sc)
                    return 0

                lax.fori_loop(0, GB_GAT, scale_row, 0)
                pltpu.sync_copy(gbuf, acc.at[bidx], add=True)
                return 0

            lax.fori_loop(0, nb, flush_body, 0)
            return 0

        lax.fori_loop(0, SCHUNK, chunk_body, 0)
        plsc.subcore_barrier()
        pltpu.sync_copy(
            acc.at[pl.ds(s * rows_per, rows_per)],
            num_hbm.at[pl.ds(lo * SLAB_H + s * rows_per, rows_per)])
        plsc.subcore_barrier()


_TB = 1024
_GRID = NPAD // _TB


def _tca_body(z_ref, w1_ref, b1_ref, w2_ref, b2_ref, wg_ref, deg_ref,
              xgs_ref, dinv_ref):
    x1 = jnp.maximum(
        jnp.dot(z_ref[...], w1_ref[...], preferred_element_type=_f32)
        + b1_ref[...], 0.0)
    x2 = jnp.maximum(
        jnp.dot(x1, w2_ref[...], preferred_element_type=_f32)
        + b2_ref[...], 0.0)
    xg = jnp.dot(x2, wg_ref[...], preferred_element_type=_f32)
    dinv = lax.rsqrt(jnp.maximum(deg_ref[...], 1.0))
    xgs_ref[...] = xg * dinv
    dinv_ref[...] = dinv


def _tcb_body(agg_ref, xgs_ref, dinv_ref, bg_ref, wa_ref, asv_ref, adv_ref,
              xh_ref, asrc_ref, adst_ref):
    x3 = jnp.maximum(
        dinv_ref[...] * (agg_ref[...] + xgs_ref[...]) + bg_ref[...], 0.0)
    xh = jnp.dot(x3, wa_ref[...], preferred_element_type=_f32)
    xh_ref[...] = xh
    asv = asv_ref[...]
    adv = adv_ref[...]
    a0 = jnp.sum(xh[:, :C] * asv[0:1, :], axis=1, keepdims=True)
    a1 = jnp.sum(xh[:, C:] * asv[1:2, :], axis=1, keepdims=True)
    asrc_ref[...] = jnp.concatenate([a0, a1], axis=1)
    d0 = jnp.sum(xh[:, :C] * adv[0:1, :], axis=1, keepdims=True)
    d1 = jnp.sum(xh[:, C:] * adv[1:2, :], axis=1, keepdims=True)
    adst_ref[...] = jnp.concatenate([d0, d1], axis=1)


def _tcc_body(num_ref, xh_ref, wself_ref, den_ref, ba_ref, out_ref):
    ws = wself_ref[...]
    dn = den_ref[...]
    scale = jnp.concatenate(
        [jnp.broadcast_to(ws[:, 0:1], (_TB, C)),
         jnp.broadcast_to(ws[:, 1:2], (_TB, C))], axis=1)
    dwide = jnp.concatenate(
        [jnp.broadcast_to(dn[:, 0:1], (_TB, C)),
         jnp.broadcast_to(dn[:, 1:2], (_TB, C))], axis=1)
    out_ref[...] = (num_ref[...] + scale * xh_ref[...]) / (
        dwide + 1e-16) + ba_ref[...]


def _row_spec(cols):
    return pl.BlockSpec((_TB, cols), lambda i: (i, 0))


def _full_spec(shape):
    return pl.BlockSpec(shape, lambda i: tuple(0 for _ in shape))


def kernel(z, edge_index, W1, b1, W2, b2, Wg, bg, Wa, att_src, att_dst, ba):
    row = edge_index[0]
    col = edge_index[1]
    zp = jnp.pad(z, ((0, NPAD - N), (0, 0)))

    deg = _s1_deg(col)

    xgs, dinv = pl.pallas_call(
        _tca_body,
        grid=(_GRID,),
        in_specs=[
            _row_spec(64), _full_spec((64, 128)), _full_spec((1, 128)),
            _full_spec((128, C)), _full_spec((1, C)),
            _full_spec((C, C)), _row_spec(1),
        ],
        out_specs=[_row_spec(C), _row_spec(1)],
        out_shape=[
            jax.ShapeDtypeStruct((NPAD, C), _f32),
            jax.ShapeDtypeStruct((NPAD, 1), _f32),
        ],
    )(zp, W1, b1.reshape(1, 128), W2, b2.reshape(1, C), Wg,
      deg.reshape(NPAD, 1))

    agg = _s2_gcn(
        xgs.reshape(NPAD * SLAB_G, 128), row, col
    ).reshape(NPAD, C)

    xh, asrc, adst = pl.pallas_call(
        _tcb_body,
        grid=(_GRID,),
        in_specs=[
            _row_spec(C), _row_spec(C), _row_spec(1), _full_spec((1, C)),
            _full_spec((C, H * C)), _full_spec((H, C)),
            _full_spec((H, C)),
        ],
        out_specs=[_row_spec(H * C), _row_spec(H), _row_spec(H)],
        out_shape=[
            jax.ShapeDtypeStruct((NPAD, H * C), _f32),
            jax.ShapeDtypeStruct((NPAD, H), _f32),
            jax.ShapeDtypeStruct((NPAD, H), _f32),
        ],
    )(agg, xgs, dinv, bg.reshape(1, C), Wa, att_src, att_dst)

    asrc_f = asrc.reshape(NPAD * 2)
    adst_f = adst.reshape(NPAD * 2)

    amax_f, den_f, wself_f = _s3_stats(row, col, asrc_f, adst_f)

    num = _s4_gat(
        xh.reshape(NPAD * SLAB_H, 128), row, col, asrc_f, adst_f, amax_f
    ).reshape(NPAD, H * C)

    out = pl.pallas_call(
        _tcc_body,
        grid=(_GRID,),
        in_specs=[
            _row_spec(H * C), _row_spec(H * C), _row_spec(H), _row_spec(H),
            _full_spec((1, H * C)),
        ],
        out_specs=_row_spec(H * C),
        out_shape=jax.ShapeDtypeStruct((NPAD, H * C), _f32),
    )(num, xh, wself_f.reshape(NPAD, H), den_f.reshape(NPAD, H),
      ba.reshape(1, H * C))

    return out[:N]


# speedup vs baseline: 8.6852x; 1.1061x over previous
import functools
import jax
import jax.numpy as jnp
from jax import lax
from jax.experimental import pallas as pl
from jax.experimental.pallas import tpu as pltpu, tpu_sc as plsc

N = 10000
NPAD = 10240
E = 160000
H = 2
C = 512

NC = 2
NS = 16
NW = NC * NS
OWN = NPAD // NW
ECH = 2000
NCHUNK = E // ECH
STRIPE = E // NS
SCHUNK = STRIPE // ECH

GCN_CH = 2560
GCN_PASS = NPAD // NC // GCN_CH
GAT_CH = 640
GAT_PASS = NPAD // NC // GAT_CH
GB_GCN = 32
GB_GAT = 16

_mesh = functools.partial(
    plsc.VectorSubcoreMesh, core_axis_name="c", subcore_axis_name="s"
)
_SC_PARAMS = pltpu.CompilerParams(needs_layout_passes=False)

_f32 = jnp.float32
_i32 = jnp.int32


def _iota16():
    return lax.iota(_i32, 16)


def _lrelu(v):
    return jnp.where(v >= 0.0, v, 0.2 * v)


def _shift_right(v, d, fill):
    i16 = _iota16()
    idx = jnp.maximum(i16 - d, 0)
    g = jnp.take_along_axis(v, idx, axis=0, mode="promise_in_bounds")
    return jnp.where(i16 >= d, g, fill)


def _shift_left1(v, fill):
    i16 = _iota16()
    idx = jnp.minimum(i16 + 1, 15)
    g = jnp.take_along_axis(v, idx, axis=0, mode="promise_in_bounds")
    return jnp.where(i16 <= 14, g, fill)


@functools.partial(
    pl.kernel,
    out_type=jax.ShapeDtypeStruct((NPAD,), _f32),
    mesh=_mesh(),
    compiler_params=_SC_PARAMS,
    scratch_types=[
        pltpu.VMEM((ECH,), _i32),
        pltpu.VMEM((OWN + 16,), _f32),
    ],
)
def _s1_deg(col_hbm, deg_hbm, colb, deg):
    w = lax.axis_index("s") * NC + lax.axis_index("c")
    base = w * OWN
    for i in range(OWN // 16):
        deg[pl.ds(i * 16, 16)] = jnp.full((16,), 1.0, _f32)
    deg[pl.ds(OWN, 16)] = jnp.zeros((16,), _f32)
    ones = jnp.full((16,), 1.0, _f32)

    def chunk_body(ch, _):
        pltpu.sync_copy(col_hbm.at[pl.ds(ch * ECH, ECH)], colb)

        def vec_body(v, _):
            c16 = colb[pl.ds(v * 16, 16)]
            m = (c16 >= base) & (c16 < base + OWN)
            lc = jnp.where(m, c16 - base, OWN)
            plsc.addupdate_scatter(deg, [lc], ones, mask=m)
            return 0

        lax.fori_loop(0, ECH // 16, vec_body, 0)
        return 0

    lax.fori_loop(0, NCHUNK, chunk_body, 0)
    pltpu.sync_copy(deg.at[pl.ds(0, OWN)], deg_hbm.at[pl.ds(base, OWN)])


SLAB_G = C // 128


@functools.partial(
    pl.kernel,
    out_type=jax.ShapeDtypeStruct((NPAD * SLAB_G, 128), _f32),
    mesh=_mesh(),
    compiler_params=_SC_PARAMS,
    scratch_types=[
        pltpu.VMEM_SHARED((GCN_CH * SLAB_G + 8, 128), _f32),
        pltpu.VMEM((ECH,), _i32),
        pltpu.VMEM((ECH,), _i32),
        pltpu.VMEM((ECH + 4 * GB_GCN,), _i32),
        pltpu.VMEM((ECH + 4 * GB_GCN,), _i32),
        pltpu.VMEM((GB_GCN * SLAB_G, 128), _f32),
        pltpu.VMEM((GB_GCN * SLAB_G, 128), _f32),
        pltpu.VMEM((32, 128), _f32),
        pltpu.VMEM((GB_GCN * SLAB_G,), _i32),
        pltpu.VMEM((GB_GCN * SLAB_G,), _i32),
        pltpu.VMEM((GB_GCN * SLAB_G,), _i32),
        pltpu.VMEM((GB_GCN * SLAB_G,), _i32),
        pltpu.SemaphoreType.DMA,
        pltpu.SemaphoreType.DMA,
    ],
)
def _s2_gcn(xgs_hbm, row_hbm, col_hbm, agg_hbm,
            acc, colb, rowb, frow, flc, gbuf0, gbuf1, zbuf,
            brow0, bidx0, brow1, bidx1, sem0, sem1):
    core = lax.axis_index("c")
    s = lax.axis_index("s")
    for i in range(32):
        for j in range(8):
            zbuf[i, pl.ds(j * 16, 16)] = jnp.zeros((16,), _f32)

    for p in range(GCN_PASS):
        lo = core * (NPAD // NC) + p * GCN_CH
        rows_per = GCN_CH // NS * SLAB_G
        for i in range(rows_per // 32):
            pltpu.sync_copy(zbuf, acc.at[pl.ds(s * rows_per + i * 32, 32)])

        @pl.when(s == 0)
        def _():
            pltpu.sync_copy(zbuf.at[pl.ds(0, 8)],
                            acc.at[pl.ds(GCN_CH * SLAB_G, 8)])

        plsc.subcore_barrier()

        def chunk_body(ci, _):
            ebase = s * STRIPE + ci * ECH
            pltpu.sync_copy(col_hbm.at[pl.ds(ebase, ECH)], colb)
            pltpu.sync_copy(row_hbm.at[pl.ds(ebase, ECH)], rowb)

            def vec_body(v, cnt):
                c16 = colb[pl.ds(v * 16, 16)]
                r16 = rowb[pl.ds(v * 16, 16)]
                m = (c16 >= lo) & (c16 < lo + GCN_CH)
                plsc.store_compressed(frow.at[pl.ds(cnt, 16)], r16, mask=m)
                plsc.store_compressed(
                    flc.at[pl.ds(cnt, 16)], c16 - lo, mask=m)
                return cnt + plsc.all_reduce_population_count(m)[0]

            cnt = lax.fori_loop(0, ECH // 16, vec_body, _i32(0))
            for t in range(2 * GB_GCN // 16):
                frow[pl.ds(cnt + t * 16, 16)] = jnp.zeros((16,), _i32)
                flc[pl.ds(cnt + t * 16, 16)] = jnp.full(
                    (16,), GCN_CH, _i32)
            nb = (cnt + GB_GCN - 1) // GB_GCN

            def build(b, br, bi):
                for g in range(GB_GCN // 16):
                    r16 = frow[pl.ds(b * GB_GCN + g * 16, 16)]
                    l16 = flc[pl.ds(b * GB_GCN + g * 16, 16)]
                    for j in range(SLAB_G):
                        br[pl.ds(j * GB_GCN + g * 16, 16)] = (
                            r16 * SLAB_G + j)
                        bi[pl.ds(j * GB_GCN + g * 16, 16)] = (
                            l16 * SLAB_G + j)

            def process(br, bi, gb, sm):
                pltpu.make_async_copy(xgs_hbm.at[br], gb, sm).wait()
                pltpu.sync_copy(gb, acc.at[bi], add=True)

            build(0, brow0, bidx0)

            @pl.when(nb > 0)
            def _():
                pltpu.async_copy(xgs_hbm.at[brow0], gbuf0, sem0)

            def pair_body(q, _):
                b1 = 2 * q + 1
                build(b1, brow1, bidx1)

                @pl.when(b1 < nb)
                def _():
                    pltpu.async_copy(xgs_hbm.at[brow1], gbuf1, sem1)

                process(brow0, bidx0, gbuf0, sem0)
                build(2 * q + 2, brow0, bidx0)

                @pl.when(2 * q + 2 < nb)
                def _():
                    pltpu.async_copy(xgs_hbm.at[brow0], gbuf0, sem0)

                @pl.when(b1 < nb)
                def _():
                    process(brow1, bidx1, gbuf1, sem1)

                return 0

            lax.fori_loop(0, (nb + 1) // 2, pair_body, 0)
            return 0

        lax.fori_loop(0, SCHUNK, chunk_body, 0)
        plsc.subcore_barrier()
        pltpu.sync_copy(
            acc.at[pl.ds(s * rows_per, rows_per)],
            agg_hbm.at[pl.ds(lo * SLAB_G + s * rows_per, rows_per)])
        plsc.subcore_barrier()


_S3OWN = OWN * 2


@functools.partial(
    pl.kernel,
    out_type=(
        jax.ShapeDtypeStruct((NPAD * 2,), _f32),
        jax.ShapeDtypeStruct((NPAD * 2,), _f32),
        jax.ShapeDtypeStruct((NPAD * 2,), _f32),
    ),
    mesh=_mesh(),
    compiler_params=_SC_PARAMS,
    scratch_types=[
        pltpu.VMEM((NPAD * 2,), _f32),
        pltpu.VMEM((_S3OWN,), _f32),
        pltpu.VMEM((_S3OWN + 16,), _f32),
        pltpu.VMEM((_S3OWN,), _f32),
        pltpu.VMEM((_S3OWN + 16,), _f32),
        pltpu.VMEM((ECH,), _i32),
        pltpu.VMEM((ECH,), _i32),
        pltpu.VMEM((ECH + 32,), _i32),
        pltpu.VMEM((16,), _i32),
    ],
)
def _s3_stats(row_hbm, col_hbm, asrc_hbm, adst_hbm,
              amax_hbm, den_hbm, wself_hbm,
              asrc, adst, amax, aself, den, colb, rowb, fpk, cntb):
    w = lax.axis_index("s") * NC + lax.axis_index("c")
    base = w * OWN
    pltpu.sync_copy(asrc_hbm, asrc)
    pltpu.sync_copy(adst_hbm.at[pl.ds(base * 2, _S3OWN)], adst)
    for i in range(_S3OWN // 16):
        gidx = base * 2 + i * 16 + _iota16()
        a = plsc.load_gather(asrc, [gidx])
        b = adst[pl.ds(i * 16, 16)]
        v = _lrelu(a + b)
        aself[pl.ds(i * 16, 16)] = v
        amax[pl.ds(i * 16, 16)] = v
    amax[pl.ds(_S3OWN, 16)] = jnp.zeros((16,), _f32)

    def filter_chunk(ci, cnt_in):
        pltpu.sync_copy(col_hbm.at[pl.ds(ci * ECH, ECH)], colb)
        pltpu.sync_copy(row_hbm.at[pl.ds(ci * ECH, ECH)], rowb)

        def vec_body(v, cnt):
            c16 = colb[pl.ds(v * 16, 16)]
            r16 = rowb[pl.ds(v * 16, 16)]
            m = (c16 >= base) & (c16 < base + OWN)
            pk = r16 | ((c16 - base) << 14)
            plsc.store_compressed(fpk.at[pl.ds(cnt, 16)], pk, mask=m)
            return cnt + plsc.all_reduce_population_count(m)[0]

        return lax.fori_loop(0, ECH // 16, vec_body, cnt_in)

    def alpha16(g):
        pk = fpk[pl.ds(g * 16, 16)]
        r16 = pk & 16383
        lc16 = pk >> 14
        res = []
        for h in range(H):
            a = plsc.load_gather(asrc, [r16 * 2 + h])
            b = plsc.load_gather(adst, [jnp.minimum(lc16 * 2 + h,
                                                    _S3OWN - 1)])
            res.append((_lrelu(a + b), lc16 * 2 + h))
        return res

    def scanA(ci, _):
        cnt = filter_chunk(ci, _i32(0))
        fpk[pl.ds(cnt, 16)] = jnp.full((16,), (OWN << 14), _i32)
        ng = (cnt + 15) // 16

        def grp(g, _):
            for al, key in alpha16(g):
                key = jnp.minimum(key, _S3OWN)
                sk, sv = plsc.sort_key_val(key, al)
                fl = (sk != _shift_right(sk, 1, _i32(-1))).astype(_i32)
                mv = sv
                flc = fl
                for d in (1, 2, 4, 8):
                    mvs = _shift_right(mv, d, _f32(-1e30))
                    fls = _shift_right(flc, d, _i32(1))
                    mv = jnp.where(flc > 0, mv, jnp.maximum(mv, mvs))
                    flc = jnp.maximum(flc, fls)
                is_last = (sk != _shift_left1(sk, _i32(-1))) | (
                    _iota16() == 15)
                old = plsc.load_gather(amax, [sk])
                plsc.store_scatter(
                    amax, [sk], jnp.maximum(old, mv), mask=is_last)
            return 0

        lax.fori_loop(0, ng, grp, 0)
        return 0

    lax.fori_loop(0, NCHUNK, scanA, 0)

    for i in range(_S3OWN // 16):
        v = jnp.exp(aself[pl.ds(i * 16, 16)] - amax[pl.ds(i * 16, 16)])
        aself[pl.ds(i * 16, 16)] = v
        den[pl.ds(i * 16, 16)] = v
    den[pl.ds(_S3OWN, 16)] = jnp.zeros((16,), _f32)

    def scanB(ci, _):
        cnt = filter_chunk(ci, _i32(0))
        fpk[pl.ds(cnt, 16)] = jnp.full((16,), (OWN << 14), _i32)
        ng = (cnt + 15) // 16

        def grp(g, _):
            for al, key in alpha16(g):
                key = jnp.minimum(key, _S3OWN)
                mx = plsc.load_gather(amax, [key])
                wv = jnp.exp(al - mx)
                plsc.addupdate_scatter(den, [key], wv)
            return 0

        lax.fori_loop(0, ng, grp, 0)
        return 0

    lax.fori_loop(0, NCHUNK, scanB, 0)

    pltpu.sync_copy(amax.at[pl.ds(0, _S3OWN)],
                    amax_hbm.at[pl.ds(base * 2, _S3OWN)])
    pltpu.sync_copy(den.at[pl.ds(0, _S3OWN)],
                    den_hbm.at[pl.ds(base * 2, _S3OWN)])
    pltpu.sync_copy(aself, wself_hbm.at[pl.ds(base * 2, _S3OWN)])


SLAB_H = H * C // 128


@functools.partial(
    pl.kernel,
    out_type=jax.ShapeDtypeStruct((NPAD * SLAB_H, 128), _f32),
    mesh=_mesh(),
    compiler_params=_SC_PARAMS,
    scratch_types=[
        pltpu.VMEM_SHARED((GAT_CH * SLAB_H + 8, 128), _f32),
        pltpu.VMEM((NPAD * 2,), _f32),
        pltpu.VMEM((GAT_CH * 2,), _f32),
        pltpu.VMEM((GAT_CH * 2,), _f32),
        pltpu.VMEM((ECH,), _i32),
        pltpu.VMEM((ECH,), _i32),
        pltpu.VMEM((ECH + 4 * GB_GAT,), _i32),
        pltpu.VMEM((ECH + 4 * GB_GAT,), _i32),
        pltpu.VMEM((GB_GAT * SLAB_H, 128), _f32),
        pltpu.VMEM((GB_GAT * SLAB_H, 128), _f32),
        pltpu.VMEM((GB_GAT,), _f32),
        pltpu.VMEM((GB_GAT,), _f32),
        pltpu.VMEM((32, 128), _f32),
        pltpu.VMEM((GB_GAT * SLAB_H,), _i32),
        pltpu.VMEM((GB_GAT * SLAB_H,), _i32),
        pltpu.VMEM((GB_GAT * SLAB_H,), _i32),
        pltpu.VMEM((GB_GAT * SLAB_H,), _i32),
        pltpu.SemaphoreType.DMA,
        pltpu.SemaphoreType.DMA,
    ],
)
def _s4_gat(xh_hbm, row_hbm, col_hbm, asrc_hbm, adst_hbm, amax_hbm,
            num_hbm, acc, asrc, adst, amx, colb, rowb, frow, flc,
            gbuf0, gbuf1, w0b, w1b, zbuf, brow0, bidx0, brow1, bidx1,
            sem0, sem1):
    core = lax.axis_index("c")
    s = lax.axis_index("s")
    pltpu.sync_copy(asrc_hbm, asrc)
    for i in range(32):
        for j in range(8):
            zbuf[i, pl.ds(j * 16, 16)] = jnp.zeros((16,), _f32)

    for p in range(GAT_PASS):
        lo = core * (NPAD // NC) + p * GAT_CH
        pltpu.sync_copy(adst_hbm.at[pl.ds(lo * 2, GAT_CH * 2)], adst)
        pltpu.sync_copy(amax_hbm.at[pl.ds(lo * 2, GAT_CH * 2)], amx)
        rows_per = GAT_CH // NS * SLAB_H
        for i in range(rows_per // 32):
            pltpu.sync_copy(zbuf, acc.at[pl.ds(s * rows_per + i * 32, 32)])

        @pl.when(s == 0)
        def _():
            pltpu.sync_copy(zbuf.at[pl.ds(0, 8)],
                            acc.at[pl.ds(GAT_CH * SLAB_H, 8)])

        plsc.subcore_barrier()

        def chunk_body(ci, _):
            ebase = s * STRIPE + ci * ECH
            pltpu.sync_copy(col_hbm.at[pl.ds(ebase, ECH)], colb)
            pltpu.sync_copy(row_hbm.at[pl.ds(ebase, ECH)], rowb)

            def vec_body(v, cnt):
                c16 = colb[pl.ds(v * 16, 16)]
                r16 = rowb[pl.ds(v * 16, 16)]
                m = (c16 >= lo) & (c16 < lo + GAT_CH)
                plsc.store_compressed(frow.at[pl.ds(cnt, 16)], r16, mask=m)
                plsc.store_compressed(
                    flc.at[pl.ds(cnt, 16)], c16 - lo, mask=m)
                return cnt + plsc.all_reduce_population_count(m)[0]

            cnt = lax.fori_loop(0, ECH // 16, vec_body, _i32(0))
            for t in range(2 * GB_GAT // 16):
                frow[pl.ds(cnt + t * 16, 16)] = jnp.zeros((16,), _i32)
                flc[pl.ds(cnt + t * 16, 16)] = jnp.full(
                    (16,), GAT_CH, _i32)
            nb = (cnt + GB_GAT - 1) // GB_GAT

            def build(b, br, bi):
                r16 = frow[pl.ds(b * GB_GAT, 16)]
                l16 = flc[pl.ds(b * GB_GAT, 16)]
                for j in range(SLAB_H):
                    br[pl.ds(j * GB_GAT, 16)] = r16 * SLAB_H + j
                    bi[pl.ds(j * GB_GAT, 16)] = l16 * SLAB_H + j

            def process(br, bi, gb, sm):
                pltpu.make_async_copy(xh_hbm.at[br], gb, sm).wait()
                r16 = lax.shift_right_logical(br[pl.ds(0, 16)], 3)
                l16 = lax.shift_right_logical(bi[pl.ds(0, 16)], 3)
                lidx = jnp.minimum(l16 * 2, GAT_CH * 2 - 2)
                for h, wb in ((0, w0b), (1, w1b)):
                    a = plsc.load_gather(asrc, [r16 * 2 + h])
                    bdd = plsc.load_gather(adst, [lidx + h])
                    mx = plsc.load_gather(amx, [lidx + h])
                    wb[...] = jnp.exp(_lrelu(a + bdd) - mx)

                def scale_row(i, _):
                    ri0 = jnp.full((16,), i, _i32)
                    s0 = plsc.load_gather(w0b, [ri0])
                    s1 = plsc.load_gather(w1b, [ri0])
                    i16 = _iota16()
                    for j in range(SLAB_H):
                        rj = jnp.full((16,), j * GB_GAT + i, _i32)
                        sc = s0 if j < SLAB_H // 2 else s1
                        for k in range(8):
                            ck = k * 16 + i16
                            v = plsc.load_gather(gb, [rj, ck])
                            plsc.store_scatter(gb, [rj, ck], v * sc)
                    return 0

                lax.fori_loop(0, GB_GAT, scale_row, 0)
                pltpu.sync_copy(gb, acc.at[bi], add=True)

            build(0, brow0, bidx0)

            @pl.when(nb > 0)
            def _():
                pltpu.async_copy(xh_hbm.at[brow0], gbuf0, sem0)

            def pair_body(q, _):
                b1 = 2 * q + 1
                build(b1, brow1, bidx1)

                @pl.when(b1 < nb)
                def _():
                    pltpu.async_copy(xh_hbm.at[brow1], gbuf1, sem1)

                process(brow0, bidx0, gbuf0, sem0)
                build(2 * q + 2, brow0, bidx0)

                @pl.when(2 * q + 2 < nb)
                def _():
                    pltpu.async_copy(xh_hbm.at[brow0], gbuf0, sem0)

                @pl.when(b1 < nb)
                def _():
                    process(brow1, bidx1, gbuf1, sem1)

                return 0

            lax.fori_loop(0, (nb + 1) // 2, pair_body, 0)
            return 0

        lax.fori_loop(0, SCHUNK, chunk_body, 0)
        plsc.subcore_barrier()
        pltpu.sync_copy(
            acc.at[pl.ds(s * rows_per, rows_per)],
            num_hbm.at[pl.ds(lo * SLAB_H + s * rows_per, rows_per)])
        plsc.subcore_barrier()


_TB = 1024
_GRID = NPAD // _TB


def _tca_body(z_ref, w1_ref, b1_ref, w2_ref, b2_ref, wg_ref, deg_ref,
              xgs_ref, dinv_ref):
    x1 = jnp.maximum(
        jnp.dot(z_ref[...], w1_ref[...], preferred_element_type=_f32)
        + b1_ref[...], 0.0)
    x2 = jnp.maximum(
        jnp.dot(x1, w2_ref[...], preferred_element_type=_f32)
        + b2_ref[...], 0.0)
    xg = jnp.dot(x2, wg_ref[...], preferred_element_type=_f32)
    dinv = lax.rsqrt(jnp.maximum(deg_ref[...], 1.0))
    xgs_ref[...] = xg * dinv
    dinv_ref[...] = dinv


def _tcb_body(agg_ref, xgs_ref, dinv_ref, bg_ref, wa_ref, asv_ref, adv_ref,
              xh_ref, asrc_ref, adst_ref):
    x3 = jnp.maximum(
        dinv_ref[...] * (agg_ref[...] + xgs_ref[...]) + bg_ref[...], 0.0)
    xh = jnp.dot(x3, wa_ref[...], preferred_element_type=_f32)
    xh_ref[...] = xh
    asv = asv_ref[...]
    adv = adv_ref[...]
    a0 = jnp.sum(xh[:, :C] * asv[0:1, :], axis=1, keepdims=True)
    a1 = jnp.sum(xh[:, C:] * asv[1:2, :], axis=1, keepdims=True)
    asrc_ref[...] = jnp.concatenate([a0, a1], axis=1)
    d0 = jnp.sum(xh[:, :C] * adv[0:1, :], axis=1, keepdims=True)
    d1 = jnp.sum(xh[:, C:] * adv[1:2, :], axis=1, keepdims=True)
    adst_ref[...] = jnp.concatenate([d0, d1], axis=1)


def _tcc_body(num_ref, xh_ref, wself_ref, den_ref, ba_ref, out_ref):
    ws = wself_ref[...]
    dn = den_ref[...]
    scale = jnp.concatenate(
        [jnp.broadcast_to(ws[:, 0:1], (_TB, C)),
         jnp.broadcast_to(ws[:, 1:2], (_TB, C))], axis=1)
    dwide = jnp.concatenate(
        [jnp.broadcast_to(dn[:, 0:1], (_TB, C)),
         jnp.broadcast_to(dn[:, 1:2], (_TB, C))], axis=1)
    out_ref[...] = (num_ref[...] + scale * xh_ref[...]) / (
        dwide + 1e-16) + ba_ref[...]


def _row_spec(cols):
    return pl.BlockSpec((_TB, cols), lambda i: (i, 0))


def _full_spec(shape):
    return pl.BlockSpec(shape, lambda i: tuple(0 for _ in shape))


def kernel(z, edge_index, W1, b1, W2, b2, Wg, bg, Wa, att_src, att_dst, ba):
    row = edge_index[0]
    col = edge_index[1]
    zp = jnp.pad(z, ((0, NPAD - N), (0, 0)))

    deg = _s1_deg(col)

    xgs, dinv = pl.pallas_call(
        _tca_body,
        grid=(_GRID,),
        in_specs=[
            _row_spec(64), _full_spec((64, 128)), _full_spec((1, 128)),
            _full_spec((128, C)), _full_spec((1, C)),
            _full_spec((C, C)), _row_spec(1),
        ],
        out_specs=[_row_spec(C), _row_spec(1)],
        out_shape=[
            jax.ShapeDtypeStruct((NPAD, C), _f32),
            jax.ShapeDtypeStruct((NPAD, 1), _f32),
        ],
    )(zp, W1, b1.reshape(1, 128), W2, b2.reshape(1, C), Wg,
      deg.reshape(NPAD, 1))

    agg = _s2_gcn(
        xgs.reshape(NPAD * SLAB_G, 128), row, col
    ).reshape(NPAD, C)

    xh, asrc, adst = pl.pallas_call(
        _tcb_body,
        grid=(_GRID,),
        in_specs=[
            _row_spec(C), _row_spec(C), _row_spec(1), _full_spec((1, C)),
            _full_spec((C, H * C)), _full_spec((H, C)),
            _full_spec((H, C)),
        ],
        out_specs=[_row_spec(H * C), _row_spec(H), _row_spec(H)],
        out_shape=[
            jax.ShapeDtypeStruct((NPAD, H * C), _f32),
            jax.ShapeDtypeStruct((NPAD, H), _f32),
            jax.ShapeDtypeStruct((NPAD, H), _f32),
        ],
    )(agg, xgs, dinv, bg.reshape(1, C), Wa, att_src, att_dst)

    asrc_f = asrc.reshape(NPAD * 2)
    adst_f = adst.reshape(NPAD * 2)

    amax_f, den_f, wself_f = _s3_stats(row, col, asrc_f, adst_f)

    num = _s4_gat(
        xh.reshape(NPAD * SLAB_H, 128), row, col, asrc_f, adst_f, amax_f
    ).reshape(NPAD, H * C)

    out = pl.pallas_call(
        _tcc_body,
        grid=(_GRID,),
        in_specs=[
            _row_spec(H * C), _row_spec(H * C), _row_spec(H), _row_spec(H),
            _full_spec((1, H * C)),
        ],
        out_specs=_row_spec(H * C),
        out_shape=jax.ShapeDtypeStruct((NPAD, H * C), _f32),
    )(num, xh, wself_f.reshape(NPAD, H), den_f.reshape(NPAD, H),
      ba.reshape(1, H * C))

    return out[:N]


# speedup vs baseline: 13.3452x; 1.5365x over previous
import functools
import jax
import jax.numpy as jnp
from jax import lax
from jax.experimental import pallas as pl
from jax.experimental.pallas import tpu as pltpu, tpu_sc as plsc

N = 10000
NPAD = 10240
E = 160000
H = 2
C = 512

NC = 2
NS = 16
NW = NC * NS
OWN = NPAD // NW
ECH = 2000
NCHUNK = E // ECH
STRIPE = E // NS
SCHUNK = STRIPE // ECH

GCN_CH = 2560
GCN_PASS = NPAD // NC // GCN_CH
GAT_CH = 640
GAT_PASS = NPAD // NC // GAT_CH
GB_GCN = 32
GB_GAT = 16

_mesh = functools.partial(
    plsc.VectorSubcoreMesh, core_axis_name="c", subcore_axis_name="s"
)
_SC_PARAMS = pltpu.CompilerParams(needs_layout_passes=False)

_f32 = jnp.float32
_i32 = jnp.int32


def _iota16():
    return lax.iota(_i32, 16)


def _lrelu(v):
    return jnp.where(v >= 0.0, v, 0.2 * v)


def _shift_right(v, d, fill):
    i16 = _iota16()
    idx = jnp.maximum(i16 - d, 0)
    g = jnp.take_along_axis(v, idx, axis=0, mode="promise_in_bounds")
    return jnp.where(i16 >= d, g, fill)


def _shift_left1(v, fill):
    i16 = _iota16()
    idx = jnp.minimum(i16 + 1, 15)
    g = jnp.take_along_axis(v, idx, axis=0, mode="promise_in_bounds")
    return jnp.where(i16 <= 14, g, fill)


@functools.partial(
    pl.kernel,
    out_type=jax.ShapeDtypeStruct((NPAD,), _f32),
    mesh=_mesh(),
    compiler_params=_SC_PARAMS,
    scratch_types=[
        pltpu.VMEM((ECH,), _i32),
        pltpu.VMEM((OWN + 16,), _f32),
    ],
)
def _s1_deg(col_hbm, deg_hbm, colb, deg):
    w = lax.axis_index("s") * NC + lax.axis_index("c")
    base = w * OWN
    for i in range(OWN // 16):
        deg[pl.ds(i * 16, 16)] = jnp.full((16,), 1.0, _f32)
    deg[pl.ds(OWN, 16)] = jnp.zeros((16,), _f32)
    ones = jnp.full((16,), 1.0, _f32)

    def chunk_body(ch, _):
        pltpu.sync_copy(col_hbm.at[pl.ds(ch * ECH, ECH)], colb)

        def vec_body(v, _):
            c16 = colb[pl.ds(v * 16, 16)]
            m = (c16 >= base) & (c16 < base + OWN)
            lc = jnp.where(m, c16 - base, OWN)
            plsc.addupdate_scatter(deg, [lc], ones, mask=m)
            return 0

        lax.fori_loop(0, ECH // 16, vec_body, 0)
        return 0

    lax.fori_loop(0, NCHUNK, chunk_body, 0)
    pltpu.sync_copy(deg.at[pl.ds(0, OWN)], deg_hbm.at[pl.ds(base, OWN)])


SLAB_G = C // 128


@functools.partial(
    pl.kernel,
    out_type=jax.ShapeDtypeStruct((NPAD * SLAB_G, 128), _f32),
    mesh=_mesh(),
    compiler_params=_SC_PARAMS,
    scratch_types=[
        pltpu.VMEM_SHARED((GCN_CH * SLAB_G + 8, 128), _f32),
        pltpu.VMEM((ECH,), _i32),
        pltpu.VMEM((ECH,), _i32),
        pltpu.VMEM((ECH + 4 * GB_GCN,), _i32),
        pltpu.VMEM((ECH + 4 * GB_GCN,), _i32),
        pltpu.VMEM((GB_GCN * SLAB_G, 128), _f32),
        pltpu.VMEM((GB_GCN * SLAB_G, 128), _f32),
        pltpu.VMEM((32, 128), _f32),
        pltpu.VMEM((GB_GCN * SLAB_G,), _i32),
        pltpu.VMEM((GB_GCN * SLAB_G,), _i32),
        pltpu.VMEM((GB_GCN * SLAB_G,), _i32),
        pltpu.VMEM((GB_GCN * SLAB_G,), _i32),
        pltpu.SemaphoreType.DMA,
        pltpu.SemaphoreType.DMA,
    ],
)
def _s2_gcn(xgs_hbm, row_hbm, col_hbm, agg_hbm,
            acc, colb, rowb, frow, flc, gbuf0, gbuf1, zbuf,
            brow0, bidx0, brow1, bidx1, sem0, sem1):
    core = lax.axis_index("c")
    s = lax.axis_index("s")
    for i in range(32):
        for j in range(8):
            zbuf[i, pl.ds(j * 16, 16)] = jnp.zeros((16,), _f32)

    for p in range(GCN_PASS):
        lo = core * (NPAD // NC) + p * GCN_CH
        rows_per = GCN_CH // NS * SLAB_G
        for i in range(rows_per // 32):
            pltpu.sync_copy(zbuf, acc.at[pl.ds(s * rows_per + i * 32, 32)])

        @pl.when(s == 0)
        def _():
            pltpu.sync_copy(zbuf.at[pl.ds(0, 8)],
                            acc.at[pl.ds(GCN_CH * SLAB_G, 8)])

        plsc.subcore_barrier()

        def chunk_body(ci, _):
            ebase = s * STRIPE + ci * ECH
            pltpu.sync_copy(col_hbm.at[pl.ds(ebase, ECH)], colb)
            pltpu.sync_copy(row_hbm.at[pl.ds(ebase, ECH)], rowb)

            def vec_body(v, cnt):
                c16 = colb[pl.ds(v * 16, 16)]
                r16 = rowb[pl.ds(v * 16, 16)]
                m = (c16 >= lo) & (c16 < lo + GCN_CH)
                plsc.store_compressed(frow.at[pl.ds(cnt, 16)], r16, mask=m)
                plsc.store_compressed(
                    flc.at[pl.ds(cnt, 16)], c16 - lo, mask=m)
                return cnt + plsc.all_reduce_population_count(m)[0]

            cnt = lax.fori_loop(0, ECH // 16, vec_body, _i32(0))
            for t in range(2 * GB_GCN // 16):
                frow[pl.ds(cnt + t * 16, 16)] = jnp.zeros((16,), _i32)
                flc[pl.ds(cnt + t * 16, 16)] = jnp.full(
                    (16,), GCN_CH, _i32)
            nb = (cnt + GB_GCN - 1) // GB_GCN

            def build(b, br, bi):
                for g in range(GB_GCN // 16):
                    r16 = frow[pl.ds(b * GB_GCN + g * 16, 16)]
                    l16 = flc[pl.ds(b * GB_GCN + g * 16, 16)]
                    for j in range(SLAB_G):
                        br[pl.ds(j * GB_GCN + g * 16, 16)] = (
                            r16 * SLAB_G + j)
                        bi[pl.ds(j * GB_GCN + g * 16, 16)] = (
                            l16 * SLAB_G + j)

            def process(br, bi, gb, sm):
                pltpu.make_async_copy(xgs_hbm.at[br], gb, sm).wait()
                pltpu.sync_copy(gb, acc.at[bi], add=True)

            build(0, brow0, bidx0)

            @pl.when(nb > 0)
            def _():
                pltpu.async_copy(xgs_hbm.at[brow0], gbuf0, sem0)

            def pair_body(q, _):
                b1 = 2 * q + 1
                build(b1, brow1, bidx1)

                @pl.when(b1 < nb)
                def _():
                    pltpu.async_copy(xgs_hbm.at[brow1], gbuf1, sem1)

                process(brow0, bidx0, gbuf0, sem0)
                build(2 * q + 2, brow0, bidx0)

                @pl.when(2 * q + 2 < nb)
                def _():
                    pltpu.async_copy(xgs_hbm.at[brow0], gbuf0, sem0)

                @pl.when(b1 < nb)
                def _():
                    process(brow1, bidx1, gbuf1, sem1)

                return 0

            lax.fori_loop(0, (nb + 1) // 2, pair_body, 0)
            return 0

        lax.fori_loop(0, SCHUNK, chunk_body, 0)
        plsc.subcore_barrier()
        pltpu.sync_copy(
            acc.at[pl.ds(s * rows_per, rows_per)],
            agg_hbm.at[pl.ds(lo * SLAB_G + s * rows_per, rows_per)])
        plsc.subcore_barrier()


_S3OWN = OWN * 2


@functools.partial(
    pl.kernel,
    out_type=(
        jax.ShapeDtypeStruct((NPAD * 2,), _f32),
        jax.ShapeDtypeStruct((NPAD * 2,), _f32),
        jax.ShapeDtypeStruct((NPAD * 2,), _f32),
    ),
    mesh=_mesh(),
    compiler_params=_SC_PARAMS,
    scratch_types=[
        pltpu.VMEM((NPAD * 2,), _f32),
        pltpu.VMEM((_S3OWN,), _f32),
        pltpu.VMEM((_S3OWN + 16,), _f32),
        pltpu.VMEM((_S3OWN,), _f32),
        pltpu.VMEM((_S3OWN + 16,), _f32),
        pltpu.VMEM((ECH,), _i32),
        pltpu.VMEM((ECH,), _i32),
        pltpu.VMEM((ECH + 32,), _i32),
        pltpu.VMEM((16,), _i32),
    ],
)
def _s3_stats(row_hbm, col_hbm, asrc_hbm, adst_hbm,
              amax_hbm, den_hbm, wself_hbm,
              asrc, adst, amax, aself, den, colb, rowb, fpk, cntb):
    w = lax.axis_index("s") * NC + lax.axis_index("c")
    base = w * OWN
    pltpu.sync_copy(asrc_hbm, asrc)
    pltpu.sync_copy(adst_hbm.at[pl.ds(base * 2, _S3OWN)], adst)
    for i in range(_S3OWN // 16):
        gidx = base * 2 + i * 16 + _iota16()
        a = plsc.load_gather(asrc, [gidx])
        b = adst[pl.ds(i * 16, 16)]
        v = _lrelu(a + b)
        aself[pl.ds(i * 16, 16)] = v
        amax[pl.ds(i * 16, 16)] = v
    amax[pl.ds(_S3OWN, 16)] = jnp.zeros((16,), _f32)

    def filter_chunk(ci, cnt_in):
        pltpu.sync_copy(col_hbm.at[pl.ds(ci * ECH, ECH)], colb)
        pltpu.sync_copy(row_hbm.at[pl.ds(ci * ECH, ECH)], rowb)

        def vec_body(v, cnt):
            c16 = colb[pl.ds(v * 16, 16)]
            r16 = rowb[pl.ds(v * 16, 16)]
            m = (c16 >= base) & (c16 < base + OWN)
            pk = r16 | ((c16 - base) << 14)
            plsc.store_compressed(fpk.at[pl.ds(cnt, 16)], pk, mask=m)
            return cnt + plsc.all_reduce_population_count(m)[0]

        return lax.fori_loop(0, ECH // 16, vec_body, cnt_in)

    def alpha16(g):
        pk = fpk[pl.ds(g * 16, 16)]
        r16 = pk & 16383
        lc16 = pk >> 14
        res = []
        for h in range(H):
            a = plsc.load_gather(asrc, [r16 * 2 + h])
            b = plsc.load_gather(adst, [jnp.minimum(lc16 * 2 + h,
                                                    _S3OWN - 1)])
            res.append((_lrelu(a + b), lc16 * 2 + h))
        return res

    def scanA(ci, _):
        cnt = filter_chunk(ci, _i32(0))
        fpk[pl.ds(cnt, 16)] = jnp.full((16,), (OWN << 14), _i32)
        ng = (cnt + 15) // 16

        def grp(g, _):
            for al, key in alpha16(g):
                key = jnp.minimum(key, _S3OWN)
                sk, sv = plsc.sort_key_val(key, al)
                fl = (sk != _shift_right(sk, 1, _i32(-1))).astype(_i32)
                mv = sv
                flc = fl
                for d in (1, 2, 4, 8):
                    mvs = _shift_right(mv, d, _f32(-1e30))
                    fls = _shift_right(flc, d, _i32(1))
                    mv = jnp.where(flc > 0, mv, jnp.maximum(mv, mvs))
                    flc = jnp.maximum(flc, fls)
                is_last = (sk != _shift_left1(sk, _i32(-1))) | (
                    _iota16() == 15)
                old = plsc.load_gather(amax, [sk])
                plsc.store_scatter(
                    amax, [sk], jnp.maximum(old, mv), mask=is_last)
            return 0

        lax.fori_loop(0, ng, grp, 0)
        return 0

    lax.fori_loop(0, NCHUNK, scanA, 0)

    for i in range(_S3OWN // 16):
        v = jnp.exp(aself[pl.ds(i * 16, 16)] - amax[pl.ds(i * 16, 16)])
        aself[pl.ds(i * 16, 16)] = v
        den[pl.ds(i * 16, 16)] = v
    den[pl.ds(_S3OWN, 16)] = jnp.zeros((16,), _f32)

    def scanB(ci, _):
        cnt = filter_chunk(ci, _i32(0))
        fpk[pl.ds(cnt, 16)] = jnp.full((16,), (OWN << 14), _i32)
        ng = (cnt + 15) // 16

        def grp(g, _):
            for al, key in alpha16(g):
                key = jnp.minimum(key, _S3OWN)
                mx = plsc.load_gather(amax, [key])
                wv = jnp.exp(al - mx)
                plsc.addupdate_scatter(den, [key], wv)
            return 0

        lax.fori_loop(0, ng, grp, 0)
        return 0

    lax.fori_loop(0, NCHUNK, scanB, 0)

    pltpu.sync_copy(amax.at[pl.ds(0, _S3OWN)],
                    amax_hbm.at[pl.ds(base * 2, _S3OWN)])
    pltpu.sync_copy(den.at[pl.ds(0, _S3OWN)],
                    den_hbm.at[pl.ds(base * 2, _S3OWN)])
    pltpu.sync_copy(aself, wself_hbm.at[pl.ds(base * 2, _S3OWN)])


SLAB_H = H * C // 128


@functools.partial(
    pl.kernel,
    out_type=jax.ShapeDtypeStruct((NPAD * SLAB_H, 128), _f32),
    mesh=_mesh(),
    compiler_params=_SC_PARAMS,
    scratch_types=[
        pltpu.VMEM_SHARED((GAT_CH * SLAB_H + 8, 128), _f32),
        pltpu.VMEM((NPAD * 2,), _f32),
        pltpu.VMEM((GAT_CH * 2,), _f32),
        pltpu.VMEM((GAT_CH * 2,), _f32),
        pltpu.VMEM((ECH,), _i32),
        pltpu.VMEM((ECH,), _i32),
        pltpu.VMEM((ECH + 4 * GB_GAT,), _i32),
        pltpu.VMEM((ECH + 4 * GB_GAT,), _i32),
        pltpu.VMEM((GB_GAT * SLAB_H, 128), _f32),
        pltpu.VMEM((GB_GAT * SLAB_H, 128), _f32),
        pltpu.VMEM((GB_GAT,), _f32),
        pltpu.VMEM((GB_GAT,), _f32),
        pltpu.VMEM((32, 128), _f32),
        pltpu.VMEM((GB_GAT * SLAB_H,), _i32),
        pltpu.VMEM((GB_GAT * SLAB_H,), _i32),
        pltpu.VMEM((GB_GAT * SLAB_H,), _i32),
        pltpu.VMEM((GB_GAT * SLAB_H,), _i32),
        pltpu.SemaphoreType.DMA,
        pltpu.SemaphoreType.DMA,
    ],
)
def _s4_gat(xh_hbm, row_hbm, col_hbm, asrc_hbm, adst_hbm, amax_hbm,
            num_hbm, acc, asrc, adst, amx, colb, rowb, frow, flc,
            gbuf0, gbuf1, w0b, w1b, zbuf, brow0, bidx0, brow1, bidx1,
            sem0, sem1):
    core = lax.axis_index("c")
    s = lax.axis_index("s")
    pltpu.sync_copy(asrc_hbm, asrc)
    for i in range(32):
        for j in range(8):
            zbuf[i, pl.ds(j * 16, 16)] = jnp.zeros((16,), _f32)

    for p in range(GAT_PASS):
        lo = core * (NPAD // NC) + p * GAT_CH
        pltpu.sync_copy(adst_hbm.at[pl.ds(lo * 2, GAT_CH * 2)], adst)
        pltpu.sync_copy(amax_hbm.at[pl.ds(lo * 2, GAT_CH * 2)], amx)
        rows_per = GAT_CH // NS * SLAB_H
        for i in range(rows_per // 32):
            pltpu.sync_copy(zbuf, acc.at[pl.ds(s * rows_per + i * 32, 32)])

        @pl.when(s == 0)
        def _():
            pltpu.sync_copy(zbuf.at[pl.ds(0, 8)],
                            acc.at[pl.ds(GAT_CH * SLAB_H, 8)])

        plsc.subcore_barrier()

        def chunk_body(ci, _):
            ebase = s * STRIPE + ci * ECH
            pltpu.sync_copy(col_hbm.at[pl.ds(ebase, ECH)], colb)
            pltpu.sync_copy(row_hbm.at[pl.ds(ebase, ECH)], rowb)

            def vec_body(v, cnt):
                c16 = colb[pl.ds(v * 16, 16)]
                r16 = rowb[pl.ds(v * 16, 16)]
                m = (c16 >= lo) & (c16 < lo + GAT_CH)
                plsc.store_compressed(frow.at[pl.ds(cnt, 16)], r16, mask=m)
                plsc.store_compressed(
                    flc.at[pl.ds(cnt, 16)], c16 - lo, mask=m)
                return cnt + plsc.all_reduce_population_count(m)[0]

            cnt = lax.fori_loop(0, ECH // 16, vec_body, _i32(0))
            for t in range(2 * GB_GAT // 16):
                frow[pl.ds(cnt + t * 16, 16)] = jnp.zeros((16,), _i32)
                flc[pl.ds(cnt + t * 16, 16)] = jnp.full(
                    (16,), GAT_CH, _i32)
            nb = (cnt + GB_GAT - 1) // GB_GAT

            def build(b, br, bi):
                r16 = frow[pl.ds(b * GB_GAT, 16)]
                l16 = flc[pl.ds(b * GB_GAT, 16)]
                for j in range(SLAB_H):
                    br[pl.ds(j * GB_GAT, 16)] = r16 * SLAB_H + j
                    bi[pl.ds(j * GB_GAT, 16)] = l16 * SLAB_H + j

            def process(br, bi, gb, sm):
                pltpu.make_async_copy(xh_hbm.at[br], gb, sm).wait()
                r16 = lax.shift_right_logical(br[pl.ds(0, 16)], 3)
                l16 = lax.shift_right_logical(bi[pl.ds(0, 16)], 3)
                lidx = jnp.minimum(l16 * 2, GAT_CH * 2 - 2)
                for h, wb in ((0, w0b), (1, w1b)):
                    a = plsc.load_gather(asrc, [r16 * 2 + h])
                    bdd = plsc.load_gather(adst, [lidx + h])
                    mx = plsc.load_gather(amx, [lidx + h])
                    wb[...] = jnp.exp(_lrelu(a + bdd) - mx)

                ws = []
                for i in range(GB_GAT):
                    ri0 = jnp.full((16,), i, _i32)
                    ws.append((plsc.load_gather(w0b, [ri0]),
                               plsc.load_gather(w1b, [ri0])))

                def scale_k(k, _):
                    for j in range(SLAB_H):
                        for i in range(GB_GAT):
                            sc = ws[i][0] if j < SLAB_H // 2 else ws[i][1]
                            r = j * GB_GAT + i
                            gb[r, pl.ds(k * 16, 16)] = (
                                gb[r, pl.ds(k * 16, 16)] * sc)
                    return 0

                lax.fori_loop(0, 8, scale_k, 0)
                pltpu.sync_copy(gb, acc.at[bi], add=True)

            build(0, brow0, bidx0)

            @pl.when(nb > 0)
            def _():
                pltpu.async_copy(xh_hbm.at[brow0], gbuf0, sem0)

            def pair_body(q, _):
                b1 = 2 * q + 1
                build(b1, brow1, bidx1)

                @pl.when(b1 < nb)
                def _():
                    pltpu.async_copy(xh_hbm.at[brow1], gbuf1, sem1)

                process(brow0, bidx0, gbuf0, sem0)
                build(2 * q + 2, brow0, bidx0)

                @pl.when(2 * q + 2 < nb)
                def _():
                    pltpu.async_copy(xh_hbm.at[brow0], gbuf0, sem0)

                @pl.when(b1 < nb)
                def _():
                    process(brow1, bidx1, gbuf1, sem1)

                return 0

            lax.fori_loop(0, (nb + 1) // 2, pair_body, 0)
            return 0

        lax.fori_loop(0, SCHUNK, chunk_body, 0)
        plsc.subcore_barrier()
        pltpu.sync_copy(
            acc.at[pl.ds(s * rows_per, rows_per)],
            num_hbm.at[pl.ds(lo * SLAB_H + s * rows_per, rows_per)])
        plsc.subcore_barrier()


_TB = 1024
_GRID = NPAD // _TB


def _tca_body(z_ref, w1_ref, b1_ref, w2_ref, b2_ref, wg_ref, deg_ref,
              xgs_ref, dinv_ref):
    x1 = jnp.maximum(
        jnp.dot(z_ref[...], w1_ref[...], preferred_element_type=_f32)
        + b1_ref[...], 0.0)
    x2 = jnp.maximum(
        jnp.dot(x1, w2_ref[...], preferred_element_type=_f32)
        + b2_ref[...], 0.0)
    xg = jnp.dot(x2, wg_ref[...], preferred_element_type=_f32)
    dinv = lax.rsqrt(jnp.maximum(deg_ref[...], 1.0))
    xgs_ref[...] = xg * dinv
    dinv_ref[...] = dinv


def _tcb_body(agg_ref, xgs_ref, dinv_ref, bg_ref, wa_ref, asv_ref, adv_ref,
              xh_ref, asrc_ref, adst_ref):
    x3 = jnp.maximum(
        dinv_ref[...] * (agg_ref[...] + xgs_ref[...]) + bg_ref[...], 0.0)
    xh = jnp.dot(x3, wa_ref[...], preferred_element_type=_f32)
    xh_ref[...] = xh
    asv = asv_ref[...]
    adv = adv_ref[...]
    a0 = jnp.sum(xh[:, :C] * asv[0:1, :], axis=1, keepdims=True)
    a1 = jnp.sum(xh[:, C:] * asv[1:2, :], axis=1, keepdims=True)
    asrc_ref[...] = jnp.concatenate([a0, a1], axis=1)
    d0 = jnp.sum(xh[:, :C] * adv[0:1, :], axis=1, keepdims=True)
    d1 = jnp.sum(xh[:, C:] * adv[1:2, :], axis=1, keepdims=True)
    adst_ref[...] = jnp.concatenate([d0, d1], axis=1)


def _tcc_body(num_ref, xh_ref, wself_ref, den_ref, ba_ref, out_ref):
    ws = wself_ref[...]
    dn = den_ref[...]
    scale = jnp.concatenate(
        [jnp.broadcast_to(ws[:, 0:1], (_TB, C)),
         jnp.broadcast_to(ws[:, 1:2], (_TB, C))], axis=1)
    dwide = jnp.concatenate(
        [jnp.broadcast_to(dn[:, 0:1], (_TB, C)),
         jnp.broadcast_to(dn[:, 1:2], (_TB, C))], axis=1)
    out_ref[...] = (num_ref[...] + scale * xh_ref[...]) / (
        dwide + 1e-16) + ba_ref[...]


def _row_spec(cols):
    return pl.BlockSpec((_TB, cols), lambda i: (i, 0))


def _full_spec(shape):
    return pl.BlockSpec(shape, lambda i: tuple(0 for _ in shape))


def kernel(z, edge_index, W1, b1, W2, b2, Wg, bg, Wa, att_src, att_dst, ba):
    row = edge_index[0]
    col = edge_index[1]
    zp = jnp.pad(z, ((0, NPAD - N), (0, 0)))

    deg = _s1_deg(col)

    xgs, dinv = pl.pallas_call(
        _tca_body,
        grid=(_GRID,),
        in_specs=[
            _row_spec(64), _full_spec((64, 128)), _full_spec((1, 128)),
            _full_spec((128, C)), _full_spec((1, C)),
            _full_spec((C, C)), _row_spec(1),
        ],
        out_specs=[_row_spec(C), _row_spec(1)],
        out_shape=[
            jax.ShapeDtypeStruct((NPAD, C), _f32),
            jax.ShapeDtypeStruct((NPAD, 1), _f32),
        ],
    )(zp, W1, b1.reshape(1, 128), W2, b2.reshape(1, C), Wg,
      deg.reshape(NPAD, 1))

    agg = _s2_gcn(
        xgs.reshape(NPAD * SLAB_G, 128), row, col
    ).reshape(NPAD, C)

    xh, asrc, adst = pl.pallas_call(
        _tcb_body,
        grid=(_GRID,),
        in_specs=[
            _row_spec(C), _row_spec(C), _row_spec(1), _full_spec((1, C)),
            _full_spec((C, H * C)), _full_spec((H, C)),
            _full_spec((H, C)),
        ],
        out_specs=[_row_spec(H * C), _row_spec(H), _row_spec(H)],
        out_shape=[
            jax.ShapeDtypeStruct((NPAD, H * C), _f32),
            jax.ShapeDtypeStruct((NPAD, H), _f32),
            jax.ShapeDtypeStruct((NPAD, H), _f32),
        ],
    )(agg, xgs, dinv, bg.reshape(1, C), Wa, att_src, att_dst)

    asrc_f = asrc.reshape(NPAD * 2)
    adst_f = adst.reshape(NPAD * 2)

    amax_f, den_f, wself_f = _s3_stats(row, col, asrc_f, adst_f)

    num = _s4_gat(
        xh.reshape(NPAD * SLAB_H, 128), row, col, asrc_f, adst_f, amax_f
    ).reshape(NPAD, H * C)

    out = pl.pallas_call(
        _tcc_body,
        grid=(_GRID,),
        in_specs=[
            _row_spec(H * C), _row_spec(H * C), _row_spec(H), _row_spec(H),
            _full_spec((1, H * C)),
        ],
        out_specs=_row_spec(H * C),
        out_shape=jax.ShapeDtypeStruct((NPAD, H * C), _f32),
    )(num, xh, wself_f.reshape(NPAD, H), den_f.reshape(NPAD, H),
      ba.reshape(1, H * C))

    return out[:N]


# speedup vs baseline: 13.4321x; 1.0065x over previous
import functools
import jax
import jax.numpy as jnp
from jax import lax
from jax.experimental import pallas as pl
from jax.experimental.pallas import tpu as pltpu, tpu_sc as plsc

N = 10000
NPAD = 10240
E = 160000
H = 2
C = 512

NC = 2
NS = 16
NW = NC * NS
OWN = NPAD // NW
ECH = 2000
NCHUNK = E // ECH
STRIPE = E // NS
SCHUNK = STRIPE // ECH

GCN_CH = 2560
GCN_PASS = NPAD // NC // GCN_CH
GAT_CH = 640
GAT_PASS = NPAD // NC // GAT_CH
GB_GCN = 32
GB_GAT = 16

_mesh = functools.partial(
    plsc.VectorSubcoreMesh, core_axis_name="c", subcore_axis_name="s"
)
_SC_PARAMS = pltpu.CompilerParams(needs_layout_passes=False)

_f32 = jnp.float32
_i32 = jnp.int32


def _iota16():
    return lax.iota(_i32, 16)


def _lrelu(v):
    return jnp.where(v >= 0.0, v, 0.2 * v)


def _shift_right(v, d, fill):
    i16 = _iota16()
    idx = jnp.maximum(i16 - d, 0)
    g = jnp.take_along_axis(v, idx, axis=0, mode="promise_in_bounds")
    return jnp.where(i16 >= d, g, fill)


def _shift_left1(v, fill):
    i16 = _iota16()
    idx = jnp.minimum(i16 + 1, 15)
    g = jnp.take_along_axis(v, idx, axis=0, mode="promise_in_bounds")
    return jnp.where(i16 <= 14, g, fill)


@functools.partial(
    pl.kernel,
    out_type=jax.ShapeDtypeStruct((NPAD,), _f32),
    mesh=_mesh(),
    compiler_params=_SC_PARAMS,
    scratch_types=[
        pltpu.VMEM((ECH,), _i32),
        pltpu.VMEM((OWN + 16,), _f32),
    ],
)
def _s1_deg(col_hbm, deg_hbm, colb, deg):
    w = lax.axis_index("s") * NC + lax.axis_index("c")
    base = w * OWN
    for i in range(OWN // 16):
        deg[pl.ds(i * 16, 16)] = jnp.full((16,), 1.0, _f32)
    deg[pl.ds(OWN, 16)] = jnp.zeros((16,), _f32)
    ones = jnp.full((16,), 1.0, _f32)

    def chunk_body(ch, _):
        pltpu.sync_copy(col_hbm.at[pl.ds(ch * ECH, ECH)], colb)

        def vec_body(v, _):
            c16 = colb[pl.ds(v * 16, 16)]
            m = (c16 >= base) & (c16 < base + OWN)
            lc = jnp.where(m, c16 - base, OWN)
            plsc.addupdate_scatter(deg, [lc], ones, mask=m)
            return 0

        lax.fori_loop(0, ECH // 16, vec_body, 0)
        return 0

    lax.fori_loop(0, NCHUNK, chunk_body, 0)
    pltpu.sync_copy(deg.at[pl.ds(0, OWN)], deg_hbm.at[pl.ds(base, OWN)])


SLAB_G = C // 128


@functools.partial(
    pl.kernel,
    out_type=jax.ShapeDtypeStruct((NPAD * SLAB_G, 128), _f32),
    mesh=_mesh(),
    compiler_params=_SC_PARAMS,
    scratch_types=[
        pltpu.VMEM_SHARED((GCN_CH * SLAB_G + 8, 128), _f32),
        pltpu.VMEM((ECH,), _i32),
        pltpu.VMEM((ECH,), _i32),
        pltpu.VMEM((ECH + 4 * GB_GCN,), _i32),
        pltpu.VMEM((ECH + 4 * GB_GCN,), _i32),
        pltpu.VMEM((GB_GCN * SLAB_G, 128), _f32),
        pltpu.VMEM((GB_GCN * SLAB_G, 128), _f32),
        pltpu.VMEM((32, 128), _f32),
        pltpu.VMEM((GB_GCN * SLAB_G,), _i32),
        pltpu.VMEM((GB_GCN * SLAB_G,), _i32),
        pltpu.VMEM((GB_GCN * SLAB_G,), _i32),
        pltpu.VMEM((GB_GCN * SLAB_G,), _i32),
        pltpu.SemaphoreType.DMA,
        pltpu.SemaphoreType.DMA,
    ],
)
def _s2_gcn(xgs_hbm, row_hbm, col_hbm, agg_hbm,
            acc, colb, rowb, frow, flc, gbuf0, gbuf1, zbuf,
            brow0, bidx0, brow1, bidx1, sem0, sem1):
    core = lax.axis_index("c")
    s = lax.axis_index("s")
    for i in range(32):
        for j in range(8):
            zbuf[i, pl.ds(j * 16, 16)] = jnp.zeros((16,), _f32)

    for p in range(GCN_PASS):
        lo = core * (NPAD // NC) + p * GCN_CH
        rows_per = GCN_CH // NS * SLAB_G
        for i in range(rows_per // 32):
            pltpu.sync_copy(zbuf, acc.at[pl.ds(s * rows_per + i * 32, 32)])

        @pl.when(s == 0)
        def _():
            pltpu.sync_copy(zbuf.at[pl.ds(0, 8)],
                            acc.at[pl.ds(GCN_CH * SLAB_G, 8)])

        plsc.subcore_barrier()

        def chunk_body(ci, _):
            ebase = s * STRIPE + ci * ECH
            pltpu.sync_copy(col_hbm.at[pl.ds(ebase, ECH)], colb)
            pltpu.sync_copy(row_hbm.at[pl.ds(ebase, ECH)], rowb)

            def vec_body(v, cnt):
                c16 = colb[pl.ds(v * 16, 16)]
                r16 = rowb[pl.ds(v * 16, 16)]
                m = (c16 >= lo) & (c16 < lo + GCN_CH)
                plsc.store_compressed(frow.at[pl.ds(cnt, 16)], r16, mask=m)
                plsc.store_compressed(
                    flc.at[pl.ds(cnt, 16)], c16 - lo, mask=m)
                return cnt + plsc.all_reduce_population_count(m)[0]

            cnt = lax.fori_loop(0, ECH // 16, vec_body, _i32(0))
            for t in range(2 * GB_GCN // 16):
                frow[pl.ds(cnt + t * 16, 16)] = jnp.zeros((16,), _i32)
                flc[pl.ds(cnt + t * 16, 16)] = jnp.full(
                    (16,), GCN_CH, _i32)
            nb = (cnt + GB_GCN - 1) // GB_GCN

            def build(b, br, bi):
                for g in range(GB_GCN // 16):
                    r16 = frow[pl.ds(b * GB_GCN + g * 16, 16)]
                    l16 = flc[pl.ds(b * GB_GCN + g * 16, 16)]
                    for j in range(SLAB_G):
                        br[pl.ds(j * GB_GCN + g * 16, 16)] = (
                            r16 * SLAB_G + j)
                        bi[pl.ds(j * GB_GCN + g * 16, 16)] = (
                            l16 * SLAB_G + j)

            def process(br, bi, gb, sm):
                pltpu.make_async_copy(xgs_hbm.at[br], gb, sm).wait()
                pltpu.sync_copy(gb, acc.at[bi], add=True)

            build(0, brow0, bidx0)

            @pl.when(nb > 0)
            def _():
                pltpu.async_copy(xgs_hbm.at[brow0], gbuf0, sem0)

            def pair_body(q, _):
                b1 = 2 * q + 1
                build(b1, brow1, bidx1)

                @pl.when(b1 < nb)
                def _():
                    pltpu.async_copy(xgs_hbm.at[brow1], gbuf1, sem1)

                process(brow0, bidx0, gbuf0, sem0)
                build(2 * q + 2, brow0, bidx0)

                @pl.when(2 * q + 2 < nb)
                def _():
                    pltpu.async_copy(xgs_hbm.at[brow0], gbuf0, sem0)

                @pl.when(b1 < nb)
                def _():
                    process(brow1, bidx1, gbuf1, sem1)

                return 0

            lax.fori_loop(0, (nb + 1) // 2, pair_body, 0)
            return 0

        lax.fori_loop(0, SCHUNK, chunk_body, 0)
        plsc.subcore_barrier()
        pltpu.sync_copy(
            acc.at[pl.ds(s * rows_per, rows_per)],
            agg_hbm.at[pl.ds(lo * SLAB_G + s * rows_per, rows_per)])
        plsc.subcore_barrier()


_S3OWN = OWN * 2


@functools.partial(
    pl.kernel,
    out_type=(
        jax.ShapeDtypeStruct((NPAD * 2,), _f32),
        jax.ShapeDtypeStruct((NPAD * 2,), _f32),
        jax.ShapeDtypeStruct((NPAD * 2,), _f32),
    ),
    mesh=_mesh(),
    compiler_params=_SC_PARAMS,
    scratch_types=[
        pltpu.VMEM((NPAD * 2,), _f32),
        pltpu.VMEM((_S3OWN,), _f32),
        pltpu.VMEM((_S3OWN + 16,), _f32),
        pltpu.VMEM((_S3OWN,), _f32),
        pltpu.VMEM((_S3OWN + 16,), _f32),
        pltpu.VMEM((ECH,), _i32),
        pltpu.VMEM((ECH,), _i32),
        pltpu.VMEM((ECH + 32,), _i32),
        pltpu.VMEM((16,), _i32),
    ],
)
def _s3_stats(row_hbm, col_hbm, asrc_hbm, adst_hbm,
              amax_hbm, den_hbm, wself_hbm,
              asrc, adst, amax, aself, den, colb, rowb, fpk, cntb):
    w = lax.axis_index("s") * NC + lax.axis_index("c")
    base = w * OWN
    pltpu.sync_copy(asrc_hbm, asrc)
    pltpu.sync_copy(adst_hbm.at[pl.ds(base * 2, _S3OWN)], adst)
    for i in range(_S3OWN // 16):
        gidx = base * 2 + i * 16 + _iota16()
        a = plsc.load_gather(asrc, [gidx])
        b = adst[pl.ds(i * 16, 16)]
        v = _lrelu(a + b)
        aself[pl.ds(i * 16, 16)] = v
        amax[pl.ds(i * 16, 16)] = v
    amax[pl.ds(_S3OWN, 16)] = jnp.zeros((16,), _f32)

    def filter_chunk(ci, cnt_in):
        pltpu.sync_copy(col_hbm.at[pl.ds(ci * ECH, ECH)], colb)
        pltpu.sync_copy(row_hbm.at[pl.ds(ci * ECH, ECH)], rowb)

        def vec_body(v, cnt):
            c16 = colb[pl.ds(v * 16, 16)]
            r16 = rowb[pl.ds(v * 16, 16)]
            m = (c16 >= base) & (c16 < base + OWN)
            pk = r16 | ((c16 - base) << 14)
            plsc.store_compressed(fpk.at[pl.ds(cnt, 16)], pk, mask=m)
            return cnt + plsc.all_reduce_population_count(m)[0]

        return lax.fori_loop(0, ECH // 16, vec_body, cnt_in)

    def alpha16(g):
        pk = fpk[pl.ds(g * 16, 16)]
        r16 = pk & 16383
        lc16 = pk >> 14
        res = []
        for h in range(H):
            a = plsc.load_gather(asrc, [r16 * 2 + h])
            b = plsc.load_gather(adst, [jnp.minimum(lc16 * 2 + h,
                                                    _S3OWN - 1)])
            res.append((_lrelu(a + b), lc16 * 2 + h))
        return res

    def scanA(ci, _):
        cnt = filter_chunk(ci, _i32(0))
        fpk[pl.ds(cnt, 16)] = jnp.full((16,), (OWN << 14), _i32)
        ng = (cnt + 15) // 16

        def grp(g, _):
            for al, key in alpha16(g):
                key = jnp.minimum(key, _S3OWN)
                sk, sv = plsc.sort_key_val(key, al)
                fl = (sk != _shift_right(sk, 1, _i32(-1))).astype(_i32)
                mv = sv
                flc = fl
                for d in (1, 2, 4, 8):
                    mvs = _shift_right(mv, d, _f32(-1e30))
                    fls = _shift_right(flc, d, _i32(1))
                    mv = jnp.where(flc > 0, mv, jnp.maximum(mv, mvs))
                    flc = jnp.maximum(flc, fls)
                is_last = (sk != _shift_left1(sk, _i32(-1))) | (
                    _iota16() == 15)
                old = plsc.load_gather(amax, [sk])
                plsc.store_scatter(
                    amax, [sk], jnp.maximum(old, mv), mask=is_last)
            return 0

        lax.fori_loop(0, ng, grp, 0)
        return 0

    lax.fori_loop(0, NCHUNK, scanA, 0)

    for i in range(_S3OWN // 16):
        v = jnp.exp(aself[pl.ds(i * 16, 16)] - amax[pl.ds(i * 16, 16)])
        aself[pl.ds(i * 16, 16)] = v
        den[pl.ds(i * 16, 16)] = v
    den[pl.ds(_S3OWN, 16)] = jnp.zeros((16,), _f32)

    def scanB(ci, _):
        cnt = filter_chunk(ci, _i32(0))
        fpk[pl.ds(cnt, 16)] = jnp.full((16,), (OWN << 14), _i32)
        ng = (cnt + 15) // 16

        def grp(g, _):
            for al, key in alpha16(g):
                key = jnp.minimum(key, _S3OWN)
                mx = plsc.load_gather(amax, [key])
                wv = jnp.exp(al - mx)
                plsc.addupdate_scatter(den, [key], wv)
            return 0

        lax.fori_loop(0, ng, grp, 0)
        return 0

    lax.fori_loop(0, NCHUNK, scanB, 0)

    pltpu.sync_copy(amax.at[pl.ds(0, _S3OWN)],
                    amax_hbm.at[pl.ds(base * 2, _S3OWN)])
    pltpu.sync_copy(den.at[pl.ds(0, _S3OWN)],
                    den_hbm.at[pl.ds(base * 2, _S3OWN)])
    pltpu.sync_copy(aself, wself_hbm.at[pl.ds(base * 2, _S3OWN)])


SLAB_H = H * C // 128


@functools.partial(
    pl.kernel,
    out_type=jax.ShapeDtypeStruct((NPAD * SLAB_H, 128), _f32),
    mesh=_mesh(),
    compiler_params=_SC_PARAMS,
    scratch_types=[
        pltpu.VMEM_SHARED((GAT_CH * SLAB_H + 8, 128), _f32),
        pltpu.VMEM((NPAD * 2,), _f32),
        pltpu.VMEM((GAT_CH * 2,), _f32),
        pltpu.VMEM((GAT_CH * 2,), _f32),
        pltpu.VMEM((ECH,), _i32),
        pltpu.VMEM((ECH,), _i32),
        pltpu.VMEM((ECH + 4 * GB_GAT,), _i32),
        pltpu.VMEM((ECH + 4 * GB_GAT,), _i32),
        pltpu.VMEM((GB_GAT * SLAB_H, 128), _f32),
        pltpu.VMEM((GB_GAT * SLAB_H, 128), _f32),
        pltpu.VMEM((32, 128), _f32),
        pltpu.VMEM((GB_GAT * SLAB_H,), _i32),
        pltpu.VMEM((GB_GAT * SLAB_H,), _i32),
        pltpu.VMEM((GB_GAT * SLAB_H,), _i32),
        pltpu.VMEM((GB_GAT * SLAB_H,), _i32),
        pltpu.SemaphoreType.DMA,
        pltpu.SemaphoreType.DMA,
    ],
)
def _s4_gat(xh_hbm, row_hbm, col_hbm, asrc_hbm, adst_hbm, amax_hbm,
            num_hbm, acc, asrc, adst, amx, colb, rowb, frow, flc,
            gbuf0, gbuf1, zbuf, brow0, bidx0, brow1, bidx1,
            sem0, sem1):
    core = lax.axis_index("c")
    s = lax.axis_index("s")
    pltpu.sync_copy(asrc_hbm, asrc)
    for i in range(32):
        for j in range(8):
            zbuf[i, pl.ds(j * 16, 16)] = jnp.zeros((16,), _f32)

    for p in range(GAT_PASS):
        lo = core * (NPAD // NC) + p * GAT_CH
        pltpu.sync_copy(adst_hbm.at[pl.ds(lo * 2, GAT_CH * 2)], adst)
        pltpu.sync_copy(amax_hbm.at[pl.ds(lo * 2, GAT_CH * 2)], amx)
        rows_per = GAT_CH // NS * SLAB_H
        for i in range(rows_per // 32):
            pltpu.sync_copy(zbuf, acc.at[pl.ds(s * rows_per + i * 32, 32)])

        @pl.when(s == 0)
        def _():
            pltpu.sync_copy(zbuf.at[pl.ds(0, 8)],
                            acc.at[pl.ds(GAT_CH * SLAB_H, 8)])

        plsc.subcore_barrier()

        def chunk_body(ci, _):
            ebase = s * STRIPE + ci * ECH
            pltpu.sync_copy(col_hbm.at[pl.ds(ebase, ECH)], colb)
            pltpu.sync_copy(row_hbm.at[pl.ds(ebase, ECH)], rowb)

            def vec_body(v, cnt):
                c16 = colb[pl.ds(v * 16, 16)]
                r16 = rowb[pl.ds(v * 16, 16)]
                m = (c16 >= lo) & (c16 < lo + GAT_CH)
                plsc.store_compressed(frow.at[pl.ds(cnt, 16)], r16, mask=m)
                plsc.store_compressed(
                    flc.at[pl.ds(cnt, 16)], c16 - lo, mask=m)
                return cnt + plsc.all_reduce_population_count(m)[0]

            cnt = lax.fori_loop(0, ECH // 16, vec_body, _i32(0))
            for t in range(2 * GB_GAT // 16):
                frow[pl.ds(cnt + t * 16, 16)] = jnp.zeros((16,), _i32)
                flc[pl.ds(cnt + t * 16, 16)] = jnp.full(
                    (16,), GAT_CH, _i32)
            nb = (cnt + GB_GAT - 1) // GB_GAT

            def build(b, br, bi):
                r16 = frow[pl.ds(b * GB_GAT, 16)]
                l16 = flc[pl.ds(b * GB_GAT, 16)]
                for j in range(SLAB_H):
                    br[pl.ds(j * GB_GAT, 16)] = r16 * SLAB_H + j
                    bi[pl.ds(j * GB_GAT, 16)] = l16 * SLAB_H + j

            def process(br, bi, gb, sm):
                pltpu.make_async_copy(xh_hbm.at[br], gb, sm).wait()
                r16 = lax.shift_right_logical(br[pl.ds(0, 16)], 3)
                l16 = lax.shift_right_logical(bi[pl.ds(0, 16)], 3)
                lidx = jnp.minimum(l16 * 2, GAT_CH * 2 - 2)
                wreg = []
                for h in range(H):
                    a = plsc.load_gather(asrc, [r16 * 2 + h])
                    bdd = plsc.load_gather(adst, [lidx + h])
                    mx = plsc.load_gather(amx, [lidx + h])
                    wreg.append(jnp.exp(_lrelu(a + bdd) - mx))

                def _splat(v, i):
                    return jnp.take_along_axis(
                        v, jnp.full((16,), i, _i32), axis=0,
                        mode="promise_in_bounds")

                ws = []
                for i in range(GB_GAT):
                    ws.append((_splat(wreg[0], i), _splat(wreg[1], i)))

                def scale_k(k, _):
                    for j in range(SLAB_H):
                        for i in range(GB_GAT):
                            sc = ws[i][0] if j < SLAB_H // 2 else ws[i][1]
                            r = j * GB_GAT + i
                            gb[r, pl.ds(k * 16, 16)] = (
                                gb[r, pl.ds(k * 16, 16)] * sc)
                    return 0

                lax.fori_loop(0, 8, scale_k, 0)
                pltpu.sync_copy(gb, acc.at[bi], add=True)

            build(0, brow0, bidx0)

            @pl.when(nb > 0)
            def _():
                pltpu.async_copy(xh_hbm.at[brow0], gbuf0, sem0)

            def pair_body(q, _):
                b1 = 2 * q + 1
                build(b1, brow1, bidx1)

                @pl.when(b1 < nb)
                def _():
                    pltpu.async_copy(xh_hbm.at[brow1], gbuf1, sem1)

                process(brow0, bidx0, gbuf0, sem0)
                build(2 * q + 2, brow0, bidx0)

                @pl.when(2 * q + 2 < nb)
                def _():
                    pltpu.async_copy(xh_hbm.at[brow0], gbuf0, sem0)

                @pl.when(b1 < nb)
                def _():
                    process(brow1, bidx1, gbuf1, sem1)

                return 0

            lax.fori_loop(0, (nb + 1) // 2, pair_body, 0)
            return 0

        lax.fori_loop(0, SCHUNK, chunk_body, 0)
        plsc.subcore_barrier()
        pltpu.sync_copy(
            acc.at[pl.ds(s * rows_per, rows_per)],
            num_hbm.at[pl.ds(lo * SLAB_H + s * rows_per, rows_per)])
        plsc.subcore_barrier()


_TB = 1024
_GRID = NPAD // _TB


def _tca_body(z_ref, w1_ref, b1_ref, w2_ref, b2_ref, wg_ref, deg_ref,
              xgs_ref, dinv_ref):
    x1 = jnp.maximum(
        jnp.dot(z_ref[...], w1_ref[...], preferred_element_type=_f32)
        + b1_ref[...], 0.0)
    x2 = jnp.maximum(
        jnp.dot(x1, w2_ref[...], preferred_element_type=_f32)
        + b2_ref[...], 0.0)
    xg = jnp.dot(x2, wg_ref[...], preferred_element_type=_f32)
    dinv = lax.rsqrt(jnp.maximum(deg_ref[...], 1.0))
    xgs_ref[...] = xg * dinv
    dinv_ref[...] = dinv


def _tcb_body(agg_ref, xgs_ref, dinv_ref, bg_ref, wa_ref, asv_ref, adv_ref,
              xh_ref, asrc_ref, adst_ref):
    x3 = jnp.maximum(
        dinv_ref[...] * (agg_ref[...] + xgs_ref[...]) + bg_ref[...], 0.0)
    xh = jnp.dot(x3, wa_ref[...], preferred_element_type=_f32)
    xh_ref[...] = xh
    asv = asv_ref[...]
    adv = adv_ref[...]
    a0 = jnp.sum(xh[:, :C] * asv[0:1, :], axis=1, keepdims=True)
    a1 = jnp.sum(xh[:, C:] * asv[1:2, :], axis=1, keepdims=True)
    asrc_ref[...] = jnp.concatenate([a0, a1], axis=1)
    d0 = jnp.sum(xh[:, :C] * adv[0:1, :], axis=1, keepdims=True)
    d1 = jnp.sum(xh[:, C:] * adv[1:2, :], axis=1, keepdims=True)
    adst_ref[...] = jnp.concatenate([d0, d1], axis=1)


def _tcc_body(num_ref, xh_ref, wself_ref, den_ref, ba_ref, out_ref):
    ws = wself_ref[...]
    dn = den_ref[...]
    scale = jnp.concatenate(
        [jnp.broadcast_to(ws[:, 0:1], (_TB, C)),
         jnp.broadcast_to(ws[:, 1:2], (_TB, C))], axis=1)
    dwide = jnp.concatenate(
        [jnp.broadcast_to(dn[:, 0:1], (_TB, C)),
         jnp.broadcast_to(dn[:, 1:2], (_TB, C))], axis=1)
    out_ref[...] = (num_ref[...] + scale * xh_ref[...]) / (
        dwide + 1e-16) + ba_ref[...]


def _row_spec(cols):
    return pl.BlockSpec((_TB, cols), lambda i: (i, 0))


def _full_spec(shape):
    return pl.BlockSpec(shape, lambda i: tuple(0 for _ in shape))


def kernel(z, edge_index, W1, b1, W2, b2, Wg, bg, Wa, att_src, att_dst, ba):
    row = edge_index[0]
    col = edge_index[1]
    zp = jnp.pad(z, ((0, NPAD - N), (0, 0)))

    deg = _s1_deg(col)

    xgs, dinv = pl.pallas_call(
        _tca_body,
        grid=(_GRID,),
        in_specs=[
            _row_spec(64), _full_spec((64, 128)), _full_spec((1, 128)),
            _full_spec((128, C)), _full_spec((1, C)),
            _full_spec((C, C)), _row_spec(1),
        ],
        out_specs=[_row_spec(C), _row_spec(1)],
        out_shape=[
            jax.ShapeDtypeStruct((NPAD, C), _f32),
            jax.ShapeDtypeStruct((NPAD, 1), _f32),
        ],
    )(zp, W1, b1.reshape(1, 128), W2, b2.reshape(1, C), Wg,
      deg.reshape(NPAD, 1))

    agg = _s2_gcn(
        xgs.reshape(NPAD * SLAB_G, 128), row, col
    ).reshape(NPAD, C)

    xh, asrc, adst = pl.pallas_call(
        _tcb_body,
        grid=(_GRID,),
        in_specs=[
            _row_spec(C), _row_spec(C), _row_spec(1), _full_spec((1, C)),
            _full_spec((C, H * C)), _full_spec((H, C)),
            _full_spec((H, C)),
        ],
        out_specs=[_row_spec(H * C), _row_spec(H), _row_spec(H)],
        out_shape=[
            jax.ShapeDtypeStruct((NPAD, H * C), _f32),
            jax.ShapeDtypeStruct((NPAD, H), _f32),
            jax.ShapeDtypeStruct((NPAD, H), _f32),
        ],
    )(agg, xgs, dinv, bg.reshape(1, C), Wa, att_src, att_dst)

    asrc_f = asrc.reshape(NPAD * 2)
    adst_f = adst.reshape(NPAD * 2)

    amax_f, den_f, wself_f = _s3_stats(row, col, asrc_f, adst_f)

    num = _s4_gat(
        xh.reshape(NPAD * SLAB_H, 128), row, col, asrc_f, adst_f, amax_f
    ).reshape(NPAD, H * C)

    out = pl.pallas_call(
        _tcc_body,
        grid=(_GRID,),
        in_specs=[
            _row_spec(H * C), _row_spec(H * C), _row_spec(H), _row_spec(H),
            _full_spec((1, H * C)),
        ],
        out_specs=_row_spec(H * C),
        out_shape=jax.ShapeDtypeStruct((NPAD, H * C), _f32),
    )(num, xh, wself_f.reshape(NPAD, H), den_f.reshape(NPAD, H),
      ba.reshape(1, H * C))

    return out[:N]
